# Initial kernel scaffold; baseline (speedup 1.0000x reference)
import jax
import jax.numpy as jnp
from jax.experimental import pallas as pl


def kernel(x, edge_index, dis, Wl1, bl1, Wr1, Wl2, bl2, Wr2, eW1a, eb1a, eW1b, eb1b, eW2a, eb2a, eW2b, eb2b, Wfc, bfc):
    raise NotImplementedError("write your pallas kernel here")


# speedup vs baseline: 9.2959x; 9.2959x over previous
import functools

import jax
import jax.numpy as jnp
from jax import lax
from jax.experimental import pallas as pl
from jax.experimental.pallas import tpu as pltpu
from jax.experimental.pallas import tpu_sc as plsc

N = 10000
E = 320000
DN = 128
DE = 16
HID = 128
NC = 2
NS = 16
NW = NC * NS
CH = 80
EPT = E // NW
NCH = EPT // CH
NPT = 624
NLO = N // CH

_mesh = plsc.VectorSubcoreMesh(core_axis_name="c", subcore_axis_name="s")
_f32 = jnp.float32


def _fill_ones(ref, rows):
    def body(r, _):
        ref[r, pl.ds(0, 16)] = jnp.ones((16,), _f32)
        return 0
    lax.fori_loop(0, rows, body, 0)


def _tile_rows(s, fn):
    fn(s * NPT, NPT)

    @pl.when(s == NS - 1)
    def _():
        fn(NS * NPT, N - NS * NPT)


@functools.partial(
    pl.kernel, mesh=_mesh,
    compiler_params=pltpu.CompilerParams(use_tc_tiling_on_sc=False),
    out_type=[
        jax.ShapeDtypeStruct((NC, N, DN), _f32),
        jax.ShapeDtypeStruct((NC, N, DE), _f32),
        jax.ShapeDtypeStruct((NC, N, DE), _f32),
    ],
    scratch_types=[
        pltpu.VMEM((CH,), jnp.int32),
        pltpu.VMEM((CH,), jnp.int32),
        pltpu.VMEM((CH, DN), _f32),
        pltpu.VMEM((CH, DE), _f32),
        pltpu.VMEM((CH, DE), _f32),
        pltpu.VMEM_SHARED((N, DN), _f32),
        pltpu.VMEM_SHARED((N, DE), _f32),
        pltpu.VMEM_SHARED((N, DE), _f32),
        pltpu.SemaphoreType.DMA,
    ],
)
def _p1(rows2d, cols2d, x, dis3, z128, z16,
        accx_o, sdis_o, cnt_o,
        rowb, colb, xrows, disb, oneb, accx_s, sdis_s, cnt_s, sem):
    c = lax.axis_index("c")
    s = lax.axis_index("s")
    wid = s * NC + c
    _fill_ones(oneb, CH)

    def init(o, n):
        pltpu.sync_copy(z128.at[pl.ds(o, n)], accx_s.at[pl.ds(o, n)])
        pltpu.sync_copy(z16.at[pl.ds(o, n)], sdis_s.at[pl.ds(o, n)])
        pltpu.sync_copy(z16.at[pl.ds(o, n)], cnt_s.at[pl.ds(o, n)])

    _tile_rows(s, init)
    plsc.subcore_barrier()

    def body(j, _):
        cid = wid * NCH + j
        pltpu.sync_copy(rows2d.at[cid], rowb)
        pltpu.sync_copy(cols2d.at[cid], colb)
        pltpu.async_copy(x.at[rowb], xrows, sem).wait()
        pltpu.sync_copy(dis3.at[cid], disb)
        pltpu.sync_copy(xrows, accx_s.at[colb], add=True)
        pltpu.sync_copy(disb, sdis_s.at[colb], add=True)
        pltpu.sync_copy(oneb, cnt_s.at[colb], add=True)
        return 0

    lax.fori_loop(0, NCH, body, 0)
    plsc.subcore_barrier()

    def writeout(o, n):
        pltpu.sync_copy(accx_s.at[pl.ds(o, n)], accx_o.at[c, pl.ds(o, n)])
        pltpu.sync_copy(sdis_s.at[pl.ds(o, n)], sdis_o.at[c, pl.ds(o, n)])
        pltpu.sync_copy(cnt_s.at[pl.ds(o, n)], cnt_o.at[c, pl.ds(o, n)])

    _tile_rows(s, writeout)


@functools.partial(
    pl.kernel, mesh=_mesh,
    compiler_params=pltpu.CompilerParams(use_tc_tiling_on_sc=False),
    out_type=jax.ShapeDtypeStruct((NC, N, HID), _f32),
    scratch_types=[
        pltpu.VMEM((CH,), jnp.int32),
        pltpu.VMEM((CH,), jnp.int32),
        pltpu.VMEM((CH, HID), _f32),
        pltpu.VMEM_SHARED((N, HID), _f32),
        pltpu.SemaphoreType.DMA,
    ],
)
def _p3a(rows2d, cols2d, h1, z128, acch_o, rowb, colb, hrows, acch_s, sem):
    c = lax.axis_index("c")
    s = lax.axis_index("s")
    wid = s * NC + c
    _tile_rows(s, lambda o, n: pltpu.sync_copy(
        z128.at[pl.ds(o, n)], acch_s.at[pl.ds(o, n)]))
    plsc.subcore_barrier()

    def body(j, _):
        cid = wid * NCH + j
        pltpu.sync_copy(rows2d.at[cid], rowb)
        pltpu.sync_copy(cols2d.at[cid], colb)
        pltpu.async_copy(h1.at[rowb], hrows, sem).wait()
        pltpu.sync_copy(hrows, acch_s.at[colb], add=True)
        return 0

    lax.fori_loop(0, NCH, body, 0)
    plsc.subcore_barrier()
    _tile_rows(s, lambda o, n: pltpu.sync_copy(
        acch_s.at[pl.ds(o, n)], acch_o.at[c, pl.ds(o, n)]))


@functools.partial(
    pl.kernel, mesh=_mesh,
    compiler_params=pltpu.CompilerParams(use_tc_tiling_on_sc=False),
    out_type=[
        jax.ShapeDtypeStruct((NC, N, HID), _f32),
        jax.ShapeDtypeStruct((NC, N, DE), _f32),
    ],
    scratch_types=[
        pltpu.VMEM((CH,), jnp.int32),
        pltpu.VMEM((CH, HID), _f32),
        pltpu.VMEM((CH, DE), _f32),
        pltpu.VMEM_SHARED((N, HID), _f32),
        pltpu.VMEM_SHARED((N, DE), _f32),
    ],
)
def _p3b(cols2d, dn13, z128, z16, accd_o, cntlo_o,
         colb, drows, oneb, accd_s, cntlo_s):
    c = lax.axis_index("c")
    s = lax.axis_index("s")
    wid = s * NC + c
    _fill_ones(oneb, CH)

    def init(o, n):
        pltpu.sync_copy(z128.at[pl.ds(o, n)], accd_s.at[pl.ds(o, n)])
        pltpu.sync_copy(z16.at[pl.ds(o, n)], cntlo_s.at[pl.ds(o, n)])

    _tile_rows(s, init)
    plsc.subcore_barrier()

    for t in range((NLO + NW - 1) // NW):
        cid = wid + NW * t

        @pl.when(cid < NLO)
        def _():
            pltpu.sync_copy(cols2d.at[cid], colb)
            pltpu.sync_copy(dn13.at[cid], drows)
            pltpu.sync_copy(drows, accd_s.at[colb], add=True)
            pltpu.sync_copy(oneb, cntlo_s.at[colb], add=True)

    plsc.subcore_barrier()

    def writeout(o, n):
        pltpu.sync_copy(accd_s.at[pl.ds(o, n)], accd_o.at[c, pl.ds(o, n)])
        pltpu.sync_copy(cntlo_s.at[pl.ds(o, n)], cntlo_o.at[c, pl.ds(o, n)])

    _tile_rows(s, writeout)


@functools.partial(
    pl.kernel, mesh=_mesh,
    out_type=jax.ShapeDtypeStruct((E,), _f32),
    compiler_params=pltpu.CompilerParams(use_tc_tiling_on_sc=False),
    scratch_types=[
        pltpu.VMEM((CH,), jnp.int32),
        pltpu.VMEM((CH,), jnp.int32),
        pltpu.VMEM((CH, 64), _f32),
        pltpu.VMEM((CH, 64), _f32),
        pltpu.VMEM((CH,), _f32),
        pltpu.VMEM((CH,), _f32),
        pltpu.SemaphoreType.DMA,
        pltpu.SemaphoreType.DMA,
    ],
)
def _p5(rows2d, cols2d, h2w, h2, dd2d, out,
        rowb, colb, rbuf, cbuf, ddb, ob, sem1, sem2):
    c = lax.axis_index("c")
    s = lax.axis_index("s")
    wid = s * NC + c

    def body(j, _):
        cid = wid * NCH + j
        pltpu.sync_copy(rows2d.at[cid], rowb)
        pltpu.sync_copy(cols2d.at[cid], colb)
        cp1 = pltpu.async_copy(h2w.at[rowb], rbuf, sem1)
        cp2 = pltpu.async_copy(h2.at[colb], cbuf, sem2)
        pltpu.sync_copy(dd2d.at[cid], ddb)
        cp1.wait()
        cp2.wait()

        lane = lax.iota(jnp.int32, 16)
        p8 = lane ^ 8
        p4 = lane ^ 4
        p2 = lane ^ 2
        p1 = lane ^ 1

        def group(g, _):
            def edot(e16, accv):
                e = g * 16 + e16
                v = rbuf[e, pl.ds(0, 16)] * cbuf[e, pl.ds(0, 16)]
                v = v + rbuf[e, pl.ds(16, 16)] * cbuf[e, pl.ds(16, 16)]
                v = v + rbuf[e, pl.ds(32, 16)] * cbuf[e, pl.ds(32, 16)]
                v = v + rbuf[e, pl.ds(48, 16)] * cbuf[e, pl.ds(48, 16)]
                v = v + v[p8]
                v = v + v[p4]
                v = v + v[p2]
                v = v + v[p1]
                return jnp.where(lane == e16, v, accv)

            accv = lax.fori_loop(0, 16, edot, jnp.zeros((16,), _f32))
            ob[pl.ds(g * 16, 16)] = accv + ddb[pl.ds(g * 16, 16)]
            return 0

        lax.fori_loop(0, CH // 16, group, 0)
        pltpu.sync_copy(ob, out.at[pl.ds(cid * CH, CH)])
        return 0

    lax.fori_loop(0, NCH, body, 0)


def _mmT(a, w):
    return lax.dot_general(a, w, (((1,), (1,)), ((), ())),
                           preferred_element_type=_f32)


def _leaky(v):
    return jnp.where(v >= 0, v, 0.01 * v)


def _tc2_body(accx2, sdis2, cnt2, x, Wl1, bl1, Wr1, eW1a, eb1a, eW1b, eb1b,
              h1_o, dn1_o):
    cnt = cnt2[0, :, 0:1] + cnt2[1, :, 0:1]
    accx = accx2[0] + accx2[1]
    invm = 1.0 / jnp.maximum(cnt, 1.0)
    inv = 1.0 / (cnt + 1.0)
    v = _mmT(accx * invm, Wl1[...]) + bl1[...][None, :] + _mmT(x[...], Wr1[...])
    h1_o[...] = _leaky(v)
    sdis = sdis2[0] + sdis2[1]
    t = _mmT(sdis, eW1a[...]) * inv + eb1a[...][None, :]
    u = _mmT(t, eW1b[...]) + eb1b[...][None, :]
    dn1_o[...] = _leaky(u)


def _tc4_body(acch2, accd2, cntlo2, cnt2, h1, dn1, c1, Wl2, bl2, Wr2,
              eW2a, eb2a, eW2b, eb2b, Wfc, bfc,
              h2w_o, h2_o, ddn_o):
    cnt = cnt2[0, :, 0:1] + cnt2[1, :, 0:1]
    invm = 1.0 / jnp.maximum(cnt, 1.0)
    inv = 1.0 / (cnt + 1.0)
    acch = acch2[0] + acch2[1]
    v = _mmT(acch * invm, Wl2[...]) + bl2[...][None, :] + _mmT(h1[...], Wr2[...])
    h2 = _leaky(v)
    accd = accd2[0] + accd2[1]
    cntlo = cntlo2[0, :, 0:1] + cntlo2[1, :, 0:1]
    S = accd + (cnt - cntlo) * c1[...]
    t = _mmT(S, eW2a[...]) * inv + eb2a[...][None, :]
    u = _mmT(t, eW2b[...]) + eb2b[...][None, :]
    dn2 = _leaky(u)
    wA = Wfc[0, 0:64][None, :]
    wB = Wfc[0, 64:128][None, :]
    h2_o[...] = h2
    h2w_o[...] = h2 * wA
    ddn_o[...] = jnp.sum(dn2 * wB, axis=1, keepdims=True) + bfc[0]


def kernel(x, edge_index, dis, Wl1, bl1, Wr1, Wl2, bl2, Wr2,
           eW1a, eb1a, eW1b, eb1b, eW2a, eb2a, eW2b, eb2b, Wfc, bfc):
    ei = edge_index.astype(jnp.int32)
    rows2d = ei[0].reshape(E // CH, CH)
    cols2d = ei[1].reshape(E // CH, CH)
    dis3 = dis.reshape(E // CH, CH, DE)
    z128 = jnp.zeros((N, DN), _f32)
    z16 = jnp.zeros((N, DE), _f32)

    accx2, sdis2, cnt2 = _p1(rows2d, cols2d, x, dis3, z128, z16)

    NB = 2000
    grid = N // NB

    def _nblk(shape):
        nd = len(shape)
        blk = shape[:-2] + (NB, shape[-1])
        idx = lambda i: (0,) * (nd - 2) + (i, 0)
        return pl.BlockSpec(blk, idx)

    def _full(shape):
        return pl.BlockSpec(shape, lambda i: (0,) * len(shape))

    h1, dn1 = pl.pallas_call(
        _tc2_body,
        grid=(grid,),
        in_specs=[_nblk((NC, N, DN)), _nblk((NC, N, DE)), _nblk((NC, N, DE)),
                  _nblk((N, DN)), _full(Wl1.shape), _full(bl1.shape),
                  _full(Wr1.shape), _full(eW1a.shape), _full(eb1a.shape),
                  _full(eW1b.shape), _full(eb1b.shape)],
        out_specs=[_nblk((N, HID)), _nblk((N, HID))],
        out_shape=[jax.ShapeDtypeStruct((N, HID), _f32),
                   jax.ShapeDtypeStruct((N, HID), _f32)],
    )(accx2, sdis2, cnt2, x, Wl1, bl1, Wr1, eW1a, eb1a, eW1b, eb1b)

    acch2 = _p3a(rows2d, cols2d, h1, z128)
    dn13 = dn1.reshape(NLO, CH, HID)
    accd2, cntlo2 = _p3b(cols2d, dn13, z128, z16)

    c1 = _leaky(eb1a @ eW1b.T + eb1b)[None, :]
    c2 = _leaky(eb2a @ eW2b.T + eb2b)
    c2s = c2 @ Wfc[0, 64:128] + bfc[0]

    h2w, h2, ddn = pl.pallas_call(
        _tc4_body,
        grid=(grid,),
        in_specs=[_nblk((NC, N, HID)), _nblk((NC, N, HID)),
                  _nblk((NC, N, DE)), _nblk((NC, N, DE)),
                  _nblk((N, HID)), _nblk((N, HID)), _full(c1.shape),
                  _full(Wl2.shape), _full(bl2.shape), _full(Wr2.shape),
                  _full(eW2a.shape), _full(eb2a.shape), _full(eW2b.shape),
                  _full(eb2b.shape), _full(Wfc.shape), _full(bfc.shape)],
        out_specs=[_nblk((N, 64)), _nblk((N, 64)), _nblk((N, 1))],
        out_shape=[jax.ShapeDtypeStruct((N, 64), _f32),
                   jax.ShapeDtypeStruct((N, 64), _f32),
                   jax.ShapeDtypeStruct((N, 1), _f32)],
    )(acch2, accd2, cntlo2, cnt2, h1, dn1, c1, Wl2, bl2, Wr2,
      eW2a, eb2a, eW2b, eb2b, Wfc, bfc)

    dd = jnp.concatenate([ddn[:, 0], jnp.full((E - N,), c2s, _f32)])
    dd2d = dd.reshape(E // CH, CH)

    out = _p5(rows2d, cols2d, h2w, h2, dd2d)
    return out[:, None]


# speedup vs baseline: 16.5338x; 1.7786x over previous
import functools

import jax
import jax.numpy as jnp
from jax import lax
from jax.experimental import pallas as pl
from jax.experimental.pallas import tpu as pltpu
from jax.experimental.pallas import tpu_sc as plsc

N = 10000
E = 320000
DN = 128
DE = 16
HID = 128
NC = 2
NS = 16
NW = NC * NS
CH = 80
EPT = E // NW
NCH = EPT // CH
NPT = 624
NLO = N // CH

_mesh = plsc.VectorSubcoreMesh(core_axis_name="c", subcore_axis_name="s")
_f32 = jnp.float32


def _fill_ones(ref, rows):
    def body(r, _):
        ref[r, pl.ds(0, 16)] = jnp.ones((16,), _f32)
        return 0
    lax.fori_loop(0, rows, body, 0)


def _tile_rows(s, fn):
    fn(s * NPT, NPT)

    @pl.when(s == NS - 1)
    def _():
        fn(NS * NPT, N - NS * NPT)


@functools.partial(
    pl.kernel, mesh=_mesh,
    compiler_params=pltpu.CompilerParams(use_tc_tiling_on_sc=False),
    out_type=[
        jax.ShapeDtypeStruct((NC, N, DN), _f32),
        jax.ShapeDtypeStruct((NC, N, DE), _f32),
        jax.ShapeDtypeStruct((NC, N, DE), _f32),
    ],
    scratch_types=[
        pltpu.VMEM((3, CH), jnp.int32),
        pltpu.VMEM((3, CH), jnp.int32),
        pltpu.VMEM((2, CH, DN), _f32),
        pltpu.VMEM((2, CH, DE), _f32),
        pltpu.VMEM((CH, DE), _f32),
        pltpu.VMEM_SHARED((N, DN), _f32),
        pltpu.VMEM_SHARED((N, DE), _f32),
        pltpu.VMEM_SHARED((N, DE), _f32),
        pltpu.SemaphoreType.DMA,
        pltpu.SemaphoreType.DMA,
        pltpu.SemaphoreType.DMA,
    ],
)
def _p1(rows2d, cols2d, x, dis3, z128, z16,
        accx_o, sdis_o, cnt_o,
        rowb, colb, xrows, disb, oneb, accx_s, sdis_s, cnt_s,
        sem_l, sem_g, sem_s):
    c = lax.axis_index("c")
    s = lax.axis_index("s")
    wid = s * NC + c
    base = wid * NCH
    _fill_ones(oneb, CH)

    def init(o, n):
        pltpu.sync_copy(z128.at[pl.ds(o, n)], accx_s.at[pl.ds(o, n)])
        pltpu.sync_copy(z16.at[pl.ds(o, n)], sdis_s.at[pl.ds(o, n)])
        pltpu.sync_copy(z16.at[pl.ds(o, n)], cnt_s.at[pl.ds(o, n)])

    _tile_rows(s, init)
    plsc.subcore_barrier()

    def ld(slot, cid):
        pltpu.async_copy(rows2d.at[cid], rowb.at[slot], sem_l)
        pltpu.async_copy(cols2d.at[cid], colb.at[slot], sem_l)

    def ldw(slot):
        pltpu.make_async_copy(rows2d.at[0], rowb.at[slot], sem_l).wait()
        pltpu.make_async_copy(cols2d.at[0], colb.at[slot], sem_l).wait()

    def sc_start(islot, dslot):
        pltpu.async_copy(xrows.at[dslot], accx_s.at[colb.at[islot]], sem_s,
                         add=True)
        pltpu.async_copy(disb.at[dslot], sdis_s.at[colb.at[islot]], sem_s,
                         add=True)
        pltpu.async_copy(oneb, cnt_s.at[colb.at[islot]], sem_s, add=True)

    def sc_wait(islot, dslot):
        pltpu.make_async_copy(xrows.at[dslot], accx_s.at[colb.at[islot]],
                              sem_s).wait()
        pltpu.make_async_copy(disb.at[dslot], sdis_s.at[colb.at[islot]],
                              sem_s).wait()
        pltpu.make_async_copy(oneb, cnt_s.at[colb.at[islot]], sem_s).wait()

    def gst(islot, dslot, cid):
        pltpu.async_copy(x.at[rowb.at[islot]], xrows.at[dslot], sem_g)
        pltpu.async_copy(dis3.at[cid], disb.at[dslot], sem_g)

    def gwt(islot, dslot):
        pltpu.make_async_copy(x.at[rowb.at[islot]], xrows.at[dslot],
                              sem_g).wait()
        pltpu.make_async_copy(dis3.at[0], disb.at[dslot], sem_g).wait()

    ld(0, base)
    ldw(0)
    gst(0, 0, base)
    ld(1, base + 1)

    def body(j, _):
        sj = lax.rem(j, 3)
        sn = lax.rem(j + 1, 3)
        sp = lax.rem(j + 2, 3)
        dj = lax.rem(j, 2)
        dn = 1 - dj

        @pl.when(j > 0)
        def _():
            sc_wait(sp, dn)

        gwt(sj, dj)

        @pl.when(j + 1 < NCH)
        def _():
            ldw(sn)
            gst(sn, dn, base + j + 1)

        @pl.when(j + 2 < NCH)
        def _():
            ld(sp, base + j + 2)

        sc_start(sj, dj)
        return 0

    lax.fori_loop(0, NCH, body, 0)
    sc_wait(lax.rem(NCH - 1, 3), lax.rem(NCH - 1, 2))
    plsc.subcore_barrier()

    def writeout(o, n):
        pltpu.sync_copy(accx_s.at[pl.ds(o, n)], accx_o.at[c, pl.ds(o, n)])
        pltpu.sync_copy(sdis_s.at[pl.ds(o, n)], sdis_o.at[c, pl.ds(o, n)])
        pltpu.sync_copy(cnt_s.at[pl.ds(o, n)], cnt_o.at[c, pl.ds(o, n)])

    _tile_rows(s, writeout)


@functools.partial(
    pl.kernel, mesh=_mesh,
    compiler_params=pltpu.CompilerParams(use_tc_tiling_on_sc=False),
    out_type=jax.ShapeDtypeStruct((NC, N, HID), _f32),
    scratch_types=[
        pltpu.VMEM((3, CH), jnp.int32),
        pltpu.VMEM((3, CH), jnp.int32),
        pltpu.VMEM((3, CH, HID), _f32),
        pltpu.VMEM_SHARED((N, HID), _f32),
        pltpu.SemaphoreType.DMA,
        pltpu.SemaphoreType.DMA,
        pltpu.SemaphoreType.DMA,
    ],
)
def _p3a(rows2d, cols2d, h1, z128, acch_o,
         rowb, colb, hrows, acch_s, sem_l, sem_g, sem_s):
    c = lax.axis_index("c")
    s = lax.axis_index("s")
    wid = s * NC + c
    base = wid * NCH
    _tile_rows(s, lambda o, n: pltpu.sync_copy(
        z128.at[pl.ds(o, n)], acch_s.at[pl.ds(o, n)]))
    plsc.subcore_barrier()

    def ld(slot, cid):
        pltpu.async_copy(rows2d.at[cid], rowb.at[slot], sem_l)
        pltpu.async_copy(cols2d.at[cid], colb.at[slot], sem_l)

    def ldw(slot):
        pltpu.make_async_copy(rows2d.at[0], rowb.at[slot], sem_l).wait()
        pltpu.make_async_copy(cols2d.at[0], colb.at[slot], sem_l).wait()

    ld(0, base)
    ldw(0)
    pltpu.async_copy(h1.at[rowb.at[0]], hrows.at[0], sem_g)
    ld(1, base + 1)

    def body(j, _):
        sj = lax.rem(j, 3)
        sn = lax.rem(j + 1, 3)
        sp = lax.rem(j + 2, 3)

        @pl.when(j > 0)
        def _():
            pltpu.make_async_copy(hrows.at[sp], acch_s.at[colb.at[sp]],
                                  sem_s).wait()

        pltpu.make_async_copy(h1.at[rowb.at[sj]], hrows.at[sj], sem_g).wait()

        @pl.when(j + 1 < NCH)
        def _():
            ldw(sn)
            pltpu.async_copy(h1.at[rowb.at[sn]], hrows.at[sn], sem_g)

        @pl.when(j + 2 < NCH)
        def _():
            ld(sp, base + j + 2)

        pltpu.async_copy(hrows.at[sj], acch_s.at[colb.at[sj]], sem_s,
                         add=True)
        return 0

    lax.fori_loop(0, NCH, body, 0)
    sl = lax.rem(NCH - 1, 3)
    pltpu.make_async_copy(hrows.at[sl], acch_s.at[colb.at[sl]], sem_s).wait()
    plsc.subcore_barrier()
    _tile_rows(s, lambda o, n: pltpu.sync_copy(
        acch_s.at[pl.ds(o, n)], acch_o.at[c, pl.ds(o, n)]))


@functools.partial(
    pl.kernel, mesh=_mesh,
    compiler_params=pltpu.CompilerParams(use_tc_tiling_on_sc=False),
    out_type=[
        jax.ShapeDtypeStruct((NC, N, HID), _f32),
        jax.ShapeDtypeStruct((NC, N, DE), _f32),
    ],
    scratch_types=[
        pltpu.VMEM((CH,), jnp.int32),
        pltpu.VMEM((CH, HID), _f32),
        pltpu.VMEM((CH, DE), _f32),
        pltpu.VMEM_SHARED((N, HID), _f32),
        pltpu.VMEM_SHARED((N, DE), _f32),
    ],
)
def _p3b(cols2d, dn13, z128, z16, accd_o, cntlo_o,
         colb, drows, oneb, accd_s, cntlo_s):
    c = lax.axis_index("c")
    s = lax.axis_index("s")
    wid = s * NC + c
    _fill_ones(oneb, CH)

    def init(o, n):
        pltpu.sync_copy(z128.at[pl.ds(o, n)], accd_s.at[pl.ds(o, n)])
        pltpu.sync_copy(z16.at[pl.ds(o, n)], cntlo_s.at[pl.ds(o, n)])

    _tile_rows(s, init)
    plsc.subcore_barrier()

    for t in range((NLO + NW - 1) // NW):
        cid = wid + NW * t

        @pl.when(cid < NLO)
        def _():
            pltpu.sync_copy(cols2d.at[cid], colb)
            pltpu.sync_copy(dn13.at[cid], drows)
            pltpu.sync_copy(drows, accd_s.at[colb], add=True)
            pltpu.sync_copy(oneb, cntlo_s.at[colb], add=True)

    plsc.subcore_barrier()

    def writeout(o, n):
        pltpu.sync_copy(accd_s.at[pl.ds(o, n)], accd_o.at[c, pl.ds(o, n)])
        pltpu.sync_copy(cntlo_s.at[pl.ds(o, n)], cntlo_o.at[c, pl.ds(o, n)])

    _tile_rows(s, writeout)


@functools.partial(
    pl.kernel, mesh=_mesh,
    out_type=jax.ShapeDtypeStruct((E,), _f32),
    compiler_params=pltpu.CompilerParams(use_tc_tiling_on_sc=False),
    scratch_types=[
        pltpu.VMEM((3, CH), jnp.int32),
        pltpu.VMEM((3, CH), jnp.int32),
        pltpu.VMEM((3, CH, 64), _f32),
        pltpu.VMEM((3, CH, 64), _f32),
        pltpu.VMEM((3, CH), _f32),
        pltpu.VMEM((3, CH), _f32),
        pltpu.SemaphoreType.DMA,
        pltpu.SemaphoreType.DMA,
        pltpu.SemaphoreType.DMA,
    ],
)
def _p5(rows2d, cols2d, h2w, h2, dd2d, out,
        rowb, colb, rbuf, cbuf, ddb, ob, sem_l, sem_g, sem_o):
    c = lax.axis_index("c")
    s = lax.axis_index("s")
    wid = s * NC + c
    base = wid * NCH

    def ld(slot, cid):
        pltpu.async_copy(rows2d.at[cid], rowb.at[slot], sem_l)
        pltpu.async_copy(cols2d.at[cid], colb.at[slot], sem_l)
        pltpu.async_copy(dd2d.at[cid], ddb.at[slot], sem_l)

    def ldw(slot):
        pltpu.make_async_copy(rows2d.at[0], rowb.at[slot], sem_l).wait()
        pltpu.make_async_copy(cols2d.at[0], colb.at[slot], sem_l).wait()
        pltpu.make_async_copy(dd2d.at[0], ddb.at[slot], sem_l).wait()

    def gst(slot):
        pltpu.async_copy(h2w.at[rowb.at[slot]], rbuf.at[slot], sem_g)
        pltpu.async_copy(h2.at[colb.at[slot]], cbuf.at[slot], sem_g)

    def gw(slot):
        pltpu.make_async_copy(h2w.at[rowb.at[slot]], rbuf.at[slot],
                              sem_g).wait()
        pltpu.make_async_copy(h2.at[colb.at[slot]], cbuf.at[slot],
                              sem_g).wait()

    ld(0, base)
    ldw(0)
    gst(0)
    ld(1, base + 1)

    lane = lax.iota(jnp.int32, 16)
    p8 = lane ^ 8
    p4 = lane ^ 4
    p2 = lane ^ 2
    p1 = lane ^ 1

    def body(j, _):
        sj = lax.rem(j, 3)
        sn = lax.rem(j + 1, 3)
        sp = lax.rem(j + 2, 3)

        @pl.when(j > 0)
        def _():
            pltpu.make_async_copy(ob.at[sp],
                                  out.at[pl.ds((base + j - 1) * CH, CH)],
                                  sem_o).wait()

        gw(sj)

        @pl.when(j + 1 < NCH)
        def _():
            ldw(sn)
            gst(sn)

        @pl.when(j + 2 < NCH)
        def _():
            ld(sp, base + j + 2)

        def group(g, _):
            def edot(e16, accv):
                e = g * 16 + e16
                v = rbuf[sj, e, pl.ds(0, 16)] * cbuf[sj, e, pl.ds(0, 16)]
                v = v + rbuf[sj, e, pl.ds(16, 16)] * cbuf[sj, e, pl.ds(16, 16)]
                v = v + rbuf[sj, e, pl.ds(32, 16)] * cbuf[sj, e, pl.ds(32, 16)]
                v = v + rbuf[sj, e, pl.ds(48, 16)] * cbuf[sj, e, pl.ds(48, 16)]
                v = v + v[p8]
                v = v + v[p4]
                v = v + v[p2]
                v = v + v[p1]
                return jnp.where(lane == e16, v, accv)

            accv = lax.fori_loop(0, 16, edot, jnp.zeros((16,), _f32))
            ob[sj, pl.ds(g * 16, 16)] = accv + ddb[sj, pl.ds(g * 16, 16)]
            return 0

        lax.fori_loop(0, CH // 16, group, 0)
        pltpu.async_copy(ob.at[sj], out.at[pl.ds((base + j) * CH, CH)], sem_o)
        return 0

    lax.fori_loop(0, NCH, body, 0)
    sl = lax.rem(NCH - 1, 3)
    pltpu.make_async_copy(ob.at[sl],
                          out.at[pl.ds((base + NCH - 1) * CH, CH)],
                          sem_o).wait()


def _mmT(a, w):
    return lax.dot_general(a, w, (((1,), (1,)), ((), ())),
                           preferred_element_type=_f32)


def _leaky(v):
    return jnp.where(v >= 0, v, 0.01 * v)


def _tc2_body(accx2, sdis2, cnt2, x, Wl1, bl1, Wr1, eW1a, eb1a, eW1b, eb1b,
              h1_o, dn1_o):
    cnt = cnt2[0, :, 0:1] + cnt2[1, :, 0:1]
    accx = accx2[0] + accx2[1]
    invm = 1.0 / jnp.maximum(cnt, 1.0)
    inv = 1.0 / (cnt + 1.0)
    v = _mmT(accx * invm, Wl1[...]) + bl1[...][None, :] + _mmT(x[...], Wr1[...])
    h1_o[...] = _leaky(v)
    sdis = sdis2[0] + sdis2[1]
    t = _mmT(sdis, eW1a[...]) * inv + eb1a[...][None, :]
    u = _mmT(t, eW1b[...]) + eb1b[...][None, :]
    dn1_o[...] = _leaky(u)


def _tc4_body(acch2, accd2, cntlo2, cnt2, h1, dn1, c1, Wl2, bl2, Wr2,
              eW2a, eb2a, eW2b, eb2b, Wfc, bfc,
              h2w_o, h2_o, ddn_o):
    cnt = cnt2[0, :, 0:1] + cnt2[1, :, 0:1]
    invm = 1.0 / jnp.maximum(cnt, 1.0)
    inv = 1.0 / (cnt + 1.0)
    acch = acch2[0] + acch2[1]
    v = _mmT(acch * invm, Wl2[...]) + bl2[...][None, :] + _mmT(h1[...], Wr2[...])
    h2 = _leaky(v)
    accd = accd2[0] + accd2[1]
    cntlo = cntlo2[0, :, 0:1] + cntlo2[1, :, 0:1]
    S = accd + (cnt - cntlo) * c1[...]
    t = _mmT(S, eW2a[...]) * inv + eb2a[...][None, :]
    u = _mmT(t, eW2b[...]) + eb2b[...][None, :]
    dn2 = _leaky(u)
    wA = Wfc[0, 0:64][None, :]
    wB = Wfc[0, 64:128][None, :]
    h2_o[...] = h2
    h2w_o[...] = h2 * wA
    ddn_o[...] = jnp.sum(dn2 * wB, axis=1, keepdims=True) + bfc[0]


def kernel(x, edge_index, dis, Wl1, bl1, Wr1, Wl2, bl2, Wr2,
           eW1a, eb1a, eW1b, eb1b, eW2a, eb2a, eW2b, eb2b, Wfc, bfc):
    ei = edge_index.astype(jnp.int32)
    rows2d = ei[0].reshape(E // CH, CH)
    cols2d = ei[1].reshape(E // CH, CH)
    dis3 = dis.reshape(E // CH, CH, DE)
    z128 = jnp.zeros((N, DN), _f32)
    z16 = jnp.zeros((N, DE), _f32)

    accx2, sdis2, cnt2 = _p1(rows2d, cols2d, x, dis3, z128, z16)

    NB = 2000
    grid = N // NB

    def _nblk(shape):
        nd = len(shape)
        blk = shape[:-2] + (NB, shape[-1])
        idx = lambda i: (0,) * (nd - 2) + (i, 0)
        return pl.BlockSpec(blk, idx)

    def _full(shape):
        return pl.BlockSpec(shape, lambda i: (0,) * len(shape))

    h1, dn1 = pl.pallas_call(
        _tc2_body,
        grid=(grid,),
        in_specs=[_nblk((NC, N, DN)), _nblk((NC, N, DE)), _nblk((NC, N, DE)),
                  _nblk((N, DN)), _full(Wl1.shape), _full(bl1.shape),
                  _full(Wr1.shape), _full(eW1a.shape), _full(eb1a.shape),
                  _full(eW1b.shape), _full(eb1b.shape)],
        out_specs=[_nblk((N, HID)), _nblk((N, HID))],
        out_shape=[jax.ShapeDtypeStruct((N, HID), _f32),
                   jax.ShapeDtypeStruct((N, HID), _f32)],
    )(accx2, sdis2, cnt2, x, Wl1, bl1, Wr1, eW1a, eb1a, eW1b, eb1b)

    acch2 = _p3a(rows2d, cols2d, h1, z128)
    dn13 = dn1.reshape(NLO, CH, HID)
    accd2, cntlo2 = _p3b(cols2d, dn13, z128, z16)

    c1 = _leaky(eb1a @ eW1b.T + eb1b)[None, :]
    c2 = _leaky(eb2a @ eW2b.T + eb2b)
    c2s = c2 @ Wfc[0, 64:128] + bfc[0]

    h2w, h2, ddn = pl.pallas_call(
        _tc4_body,
        grid=(grid,),
        in_specs=[_nblk((NC, N, HID)), _nblk((NC, N, HID)),
                  _nblk((NC, N, DE)), _nblk((NC, N, DE)),
                  _nblk((N, HID)), _nblk((N, HID)), _full(c1.shape),
                  _full(Wl2.shape), _full(bl2.shape), _full(Wr2.shape),
                  _full(eW2a.shape), _full(eb2a.shape), _full(eW2b.shape),
                  _full(eb2b.shape), _full(Wfc.shape), _full(bfc.shape)],
        out_specs=[_nblk((N, 64)), _nblk((N, 64)), _nblk((N, 1))],
        out_shape=[jax.ShapeDtypeStruct((N, 64), _f32),
                   jax.ShapeDtypeStruct((N, 64), _f32),
                   jax.ShapeDtypeStruct((N, 1), _f32)],
    )(acch2, accd2, cntlo2, cnt2, h1, dn1, c1, Wl2, bl2, Wr2,
      eW2a, eb2a, eW2b, eb2b, Wfc, bfc)

    dd = jnp.concatenate([ddn[:, 0], jnp.full((E - N,), c2s, _f32)])
    dd2d = dd.reshape(E // CH, CH)

    out = _p5(rows2d, cols2d, h2w, h2, dd2d)
    return out[:, None]


# speedup vs baseline: 18.2189x; 1.1019x over previous
import functools

import jax
import jax.numpy as jnp
from jax import lax
from jax.experimental import pallas as pl
from jax.experimental.pallas import tpu as pltpu
from jax.experimental.pallas import tpu_sc as plsc

N = 10000
E = 320000
DN = 128
DE = 16
HID = 128
NC = 2
NS = 16
NW = NC * NS
NPT = 624

CH1 = 80
NCH1 = (E // NW) // CH1
CH = 128
NCHF = 78
SPT = NCHF * CH
EXTRA0 = NW * SPT
NEX = (E - EXTRA0) // CH
NLO = N // CH
LOTAIL = N - NLO * CH

_mesh = plsc.VectorSubcoreMesh(core_axis_name="c", subcore_axis_name="s")
_f32 = jnp.float32


def _fill_ones(ref, rows):
    def body(r, _):
        ref[r, pl.ds(0, 16)] = jnp.ones((16,), _f32)
        return 0
    lax.fori_loop(0, rows, body, 0)


def _tile_rows(s, fn):
    fn(s * NPT, NPT)

    @pl.when(s == NS - 1)
    def _():
        fn(NS * NPT, N - NS * NPT)


@functools.partial(
    pl.kernel, mesh=_mesh,
    compiler_params=pltpu.CompilerParams(use_tc_tiling_on_sc=False),
    out_type=[
        jax.ShapeDtypeStruct((NC, N, DN), _f32),
        jax.ShapeDtypeStruct((NC, N, DE), _f32),
        jax.ShapeDtypeStruct((NC, N, DE), _f32),
    ],
    scratch_types=[
        pltpu.VMEM((3, CH1), jnp.int32),
        pltpu.VMEM((3, CH1), jnp.int32),
        pltpu.VMEM((2, CH1, DN), _f32),
        pltpu.VMEM((2, CH1, DE), _f32),
        pltpu.VMEM((CH1, DE), _f32),
        pltpu.VMEM_SHARED((N, DN), _f32),
        pltpu.VMEM_SHARED((N, DE), _f32),
        pltpu.VMEM_SHARED((N, DE), _f32),
        pltpu.SemaphoreType.DMA,
        pltpu.SemaphoreType.DMA,
        pltpu.SemaphoreType.DMA,
    ],
)
def _p1(rows2d, cols2d, x, dis3, z128, z16,
        accx_o, sdis_o, cnt_o,
        rowb, colb, xrows, disb, oneb, accx_s, sdis_s, cnt_s,
        sem_l, sem_g, sem_s):
    c = lax.axis_index("c")
    s = lax.axis_index("s")
    wid = s * NC + c
    base = wid * NCH1
    _fill_ones(oneb, CH1)

    def init(o, n):
        pltpu.sync_copy(z128.at[pl.ds(o, n)], accx_s.at[pl.ds(o, n)])
        pltpu.sync_copy(z16.at[pl.ds(o, n)], sdis_s.at[pl.ds(o, n)])
        pltpu.sync_copy(z16.at[pl.ds(o, n)], cnt_s.at[pl.ds(o, n)])

    _tile_rows(s, init)
    plsc.subcore_barrier()

    def ld(slot, cid):
        pltpu.async_copy(rows2d.at[cid], rowb.at[slot], sem_l)
        pltpu.async_copy(cols2d.at[cid], colb.at[slot], sem_l)

    def ldw(slot):
        pltpu.make_async_copy(rows2d.at[0], rowb.at[slot], sem_l).wait()
        pltpu.make_async_copy(cols2d.at[0], colb.at[slot], sem_l).wait()

    def sc_start(islot, dslot):
        pltpu.async_copy(xrows.at[dslot], accx_s.at[colb.at[islot]], sem_s,
                         add=True)
        pltpu.async_copy(disb.at[dslot], sdis_s.at[colb.at[islot]], sem_s,
                         add=True)
        pltpu.async_copy(oneb, cnt_s.at[colb.at[islot]], sem_s, add=True)

    def sc_wait(islot, dslot):
        pltpu.make_async_copy(xrows.at[dslot], accx_s.at[colb.at[islot]],
                              sem_s).wait()
        pltpu.make_async_copy(disb.at[dslot], sdis_s.at[colb.at[islot]],
                              sem_s).wait()
        pltpu.make_async_copy(oneb, cnt_s.at[colb.at[islot]], sem_s).wait()

    def gst(islot, dslot, cid):
        pltpu.async_copy(x.at[rowb.at[islot]], xrows.at[dslot], sem_g)
        pltpu.async_copy(dis3.at[cid], disb.at[dslot], sem_g)

    def gwt(islot, dslot):
        pltpu.make_async_copy(x.at[rowb.at[islot]], xrows.at[dslot],
                              sem_g).wait()
        pltpu.make_async_copy(dis3.at[0], disb.at[dslot], sem_g).wait()

    ld(0, base)
    ldw(0)
    gst(0, 0, base)
    ld(1, base + 1)

    def body(j, _):
        sj = lax.rem(j, 3)
        sn = lax.rem(j + 1, 3)
        sp = lax.rem(j + 2, 3)
        dj = lax.rem(j, 2)
        dn = 1 - dj

        @pl.when(j > 0)
        def _():
            sc_wait(sp, dn)

        gwt(sj, dj)

        @pl.when(j + 1 < NCH1)
        def _():
            ldw(sn)
            gst(sn, dn, base + j + 1)

        @pl.when(j + 2 < NCH1)
        def _():
            ld(sp, base + j + 2)

        sc_start(sj, dj)
        return 0

    lax.fori_loop(0, NCH1, body, 0)
    sc_wait(lax.rem(NCH1 - 1, 3), lax.rem(NCH1 - 1, 2))
    plsc.subcore_barrier()

    def writeout(o, n):
        pltpu.sync_copy(accx_s.at[pl.ds(o, n)], accx_o.at[c, pl.ds(o, n)])
        pltpu.sync_copy(sdis_s.at[pl.ds(o, n)], sdis_o.at[c, pl.ds(o, n)])
        pltpu.sync_copy(cnt_s.at[pl.ds(o, n)], cnt_o.at[c, pl.ds(o, n)])

    _tile_rows(s, writeout)


@functools.partial(
    pl.kernel, mesh=_mesh,
    compiler_params=pltpu.CompilerParams(use_tc_tiling_on_sc=False),
    out_type=[
        jax.ShapeDtypeStruct((NC, N, HID), _f32),
        jax.ShapeDtypeStruct((NC, N, HID), _f32),
    ],
    scratch_types=[
        pltpu.VMEM((3, CH), jnp.int32),
        pltpu.VMEM((3, CH), jnp.int32),
        pltpu.VMEM((2, CH, HID), _f32),
        pltpu.VMEM((LOTAIL,), jnp.int32),
        pltpu.VMEM((LOTAIL, HID), _f32),
        pltpu.VMEM_SHARED((N, HID), _f32),
        pltpu.SemaphoreType.DMA,
        pltpu.SemaphoreType.DMA,
        pltpu.SemaphoreType.DMA,
    ],
)
def _p3(rows1, cols1, h1, dn1m, z128, acch_o, accd_o,
        rowb, colb, hrows, colb16, drows16, acc_s, sem_l, sem_g, sem_s):
    c = lax.axis_index("c")
    s = lax.axis_index("s")
    wid = s * NC + c
    ebase = wid * SPT

    def zero_table(o, n):
        pltpu.sync_copy(z128.at[pl.ds(o, n)], acc_s.at[pl.ds(o, n)])

    _tile_rows(s, zero_table)
    plsc.subcore_barrier()

    def ld(slot, st):
        pltpu.async_copy(rows1.at[pl.ds(st, CH)], rowb.at[slot], sem_l)
        pltpu.async_copy(cols1.at[pl.ds(st, CH)], colb.at[slot], sem_l)

    def ldw(slot):
        pltpu.make_async_copy(rows1.at[pl.ds(0, CH)], rowb.at[slot],
                              sem_l).wait()
        pltpu.make_async_copy(cols1.at[pl.ds(0, CH)], colb.at[slot],
                              sem_l).wait()

    ld(0, ebase)
    ldw(0)
    pltpu.async_copy(h1.at[rowb.at[0]], hrows.at[0], sem_g)
    ld(1, ebase + CH)

    def body(j, _):
        sj = lax.rem(j, 3)
        sn = lax.rem(j + 1, 3)
        sp = lax.rem(j + 2, 3)
        dj = lax.rem(j, 2)
        dn = 1 - dj

        @pl.when(j > 0)
        def _():
            pltpu.make_async_copy(hrows.at[dn], acc_s.at[colb.at[sp]],
                                  sem_s).wait()

        pltpu.make_async_copy(h1.at[rowb.at[sj]], hrows.at[dj], sem_g).wait()

        @pl.when(j + 1 < NCHF)
        def _():
            ldw(sn)
            pltpu.async_copy(h1.at[rowb.at[sn]], hrows.at[dn], sem_g)

        @pl.when(j + 2 < NCHF)
        def _():
            ld(sp, ebase + (j + 2) * CH)

        pltpu.async_copy(hrows.at[dj], acc_s.at[colb.at[sj]], sem_s,
                         add=True)
        return 0

    lax.fori_loop(0, NCHF, body, 0)
    pltpu.make_async_copy(hrows.at[lax.rem(NCHF - 1, 2)],
                          acc_s.at[colb.at[lax.rem(NCHF - 1, 3)]],
                          sem_s).wait()

    @pl.when(wid < NEX)
    def _():
        st = EXTRA0 + wid * CH
        ld(0, st)
        ldw(0)
        pltpu.async_copy(h1.at[rowb.at[0]], hrows.at[0], sem_g)
        pltpu.make_async_copy(h1.at[rowb.at[0]], hrows.at[0], sem_g).wait()
        pltpu.async_copy(hrows.at[0], acc_s.at[colb.at[0]], sem_s, add=True)
        pltpu.make_async_copy(hrows.at[0], acc_s.at[colb.at[0]],
                              sem_s).wait()

    plsc.subcore_barrier()

    def wr_acch(o, n):
        pltpu.sync_copy(acc_s.at[pl.ds(o, n)], acch_o.at[c, pl.ds(o, n)])
        pltpu.sync_copy(z128.at[pl.ds(o, n)], acc_s.at[pl.ds(o, n)])

    _tile_rows(s, wr_acch)
    plsc.subcore_barrier()

    for t in range((NLO + NW - 1) // NW):
        cid = wid + NW * t

        @pl.when(cid < NLO)
        def _():
            st = cid * CH
            pltpu.sync_copy(cols1.at[pl.ds(st, CH)], colb.at[0])
            pltpu.sync_copy(dn1m.at[pl.ds(st, CH)], hrows.at[0])
            pltpu.async_copy(hrows.at[0], acc_s.at[colb.at[0]], sem_s,
                             add=True)
            pltpu.make_async_copy(hrows.at[0], acc_s.at[colb.at[0]],
                                  sem_s).wait()

    @pl.when(wid == NW - 1)
    def _():
        st = NLO * CH
        pltpu.sync_copy(cols1.at[pl.ds(st, LOTAIL)], colb16)
        pltpu.sync_copy(dn1m.at[pl.ds(st, LOTAIL)], drows16)
        pltpu.async_copy(drows16, acc_s.at[colb16], sem_s, add=True)
        pltpu.make_async_copy(drows16, acc_s.at[colb16], sem_s).wait()

    plsc.subcore_barrier()
    _tile_rows(s, lambda o, n: pltpu.sync_copy(
        acc_s.at[pl.ds(o, n)], accd_o.at[c, pl.ds(o, n)]))


@functools.partial(
    pl.kernel, mesh=_mesh,
    out_type=jax.ShapeDtypeStruct((E,), _f32),
    compiler_params=pltpu.CompilerParams(use_tc_tiling_on_sc=False),
    scratch_types=[
        pltpu.VMEM((3, CH), jnp.int32),
        pltpu.VMEM((3, CH), jnp.int32),
        pltpu.VMEM((3, CH, 64), _f32),
        pltpu.VMEM((3, CH, 64), _f32),
        pltpu.VMEM((3, CH), _f32),
        pltpu.VMEM((3, CH), _f32),
        pltpu.SemaphoreType.DMA,
        pltpu.SemaphoreType.DMA,
        pltpu.SemaphoreType.DMA,
    ],
)
def _p5(rows1, cols1, h2w, h2, dd1, out,
        rowb, colb, rbuf, cbuf, ddb, ob, sem_l, sem_g, sem_o):
    c = lax.axis_index("c")
    s = lax.axis_index("s")
    wid = s * NC + c
    ebase = wid * SPT

    def ld(slot, st):
        pltpu.async_copy(rows1.at[pl.ds(st, CH)], rowb.at[slot], sem_l)
        pltpu.async_copy(cols1.at[pl.ds(st, CH)], colb.at[slot], sem_l)
        pltpu.async_copy(dd1.at[pl.ds(st, CH)], ddb.at[slot], sem_l)

    def ldw(slot):
        pltpu.make_async_copy(rows1.at[pl.ds(0, CH)], rowb.at[slot],
                              sem_l).wait()
        pltpu.make_async_copy(cols1.at[pl.ds(0, CH)], colb.at[slot],
                              sem_l).wait()
        pltpu.make_async_copy(dd1.at[pl.ds(0, CH)], ddb.at[slot],
                              sem_l).wait()

    def gst(slot):
        pltpu.async_copy(h2w.at[rowb.at[slot]], rbuf.at[slot], sem_g)
        pltpu.async_copy(h2.at[colb.at[slot]], cbuf.at[slot], sem_g)

    def gw(slot):
        pltpu.make_async_copy(h2w.at[rowb.at[slot]], rbuf.at[slot],
                              sem_g).wait()
        pltpu.make_async_copy(h2.at[colb.at[slot]], cbuf.at[slot],
                              sem_g).wait()

    lane = lax.iota(jnp.int32, 16)
    p8 = lane ^ 8
    p4 = lane ^ 4
    p2 = lane ^ 2
    p1 = lane ^ 1

    def compute(sj):
        def group(g, _):
            def edot(e16, accv):
                e = g * 16 + e16
                v = rbuf[sj, e, pl.ds(0, 16)] * cbuf[sj, e, pl.ds(0, 16)]
                v = v + rbuf[sj, e, pl.ds(16, 16)] * cbuf[sj, e, pl.ds(16, 16)]
                v = v + rbuf[sj, e, pl.ds(32, 16)] * cbuf[sj, e, pl.ds(32, 16)]
                v = v + rbuf[sj, e, pl.ds(48, 16)] * cbuf[sj, e, pl.ds(48, 16)]
                v = v + v[p8]
                v = v + v[p4]
                v = v + v[p2]
                v = v + v[p1]
                return jnp.where(lane == e16, v, accv)

            accv = lax.fori_loop(0, 16, edot, jnp.zeros((16,), _f32))
            ob[sj, pl.ds(g * 16, 16)] = accv + ddb[sj, pl.ds(g * 16, 16)]
            return 0

        lax.fori_loop(0, CH // 16, group, 0)

    ld(0, ebase)
    ldw(0)
    gst(0)
    ld(1, ebase + CH)

    def body(j, _):
        sj = lax.rem(j, 3)
        sn = lax.rem(j + 1, 3)
        sp = lax.rem(j + 2, 3)

        @pl.when(j > 0)
        def _():
            pltpu.make_async_copy(
                ob.at[sp], out.at[pl.ds(ebase + (j - 1) * CH, CH)],
                sem_o).wait()

        gw(sj)

        @pl.when(j + 1 < NCHF)
        def _():
            ldw(sn)
            gst(sn)

        @pl.when(j + 2 < NCHF)
        def _():
            ld(sp, ebase + (j + 2) * CH)

        compute(sj)
        pltpu.async_copy(ob.at[sj], out.at[pl.ds(ebase + j * CH, CH)], sem_o)
        return 0

    lax.fori_loop(0, NCHF, body, 0)
    sl = lax.rem(NCHF - 1, 3)
    pltpu.make_async_copy(ob.at[sl],
                          out.at[pl.ds(ebase + (NCHF - 1) * CH, CH)],
                          sem_o).wait()

    @pl.when(wid < NEX)
    def _():
        st = EXTRA0 + wid * CH
        ld(0, st)
        ldw(0)
        gst(0)
        gw(0)
        compute(0)
        pltpu.async_copy(ob.at[0], out.at[pl.ds(st, CH)], sem_o)
        pltpu.make_async_copy(ob.at[0], out.at[pl.ds(st, CH)], sem_o).wait()


def _mmT(a, w):
    return lax.dot_general(a, w, (((1,), (1,)), ((), ())),
                           preferred_element_type=_f32)


def _leaky(v):
    return jnp.where(v >= 0, v, 0.01 * v)


def _tc2_body(accx2, sdis2, cnt2, x, c1, Wl1, bl1, Wr1, eW1a, eb1a, eW1b,
              eb1b, h1_o, dn1m_o):
    cnt = cnt2[0, :, 0:1] + cnt2[1, :, 0:1]
    accx = accx2[0] + accx2[1]
    invm = 1.0 / jnp.maximum(cnt, 1.0)
    inv = 1.0 / (cnt + 1.0)
    v = _mmT(accx * invm, Wl1[...]) + bl1[...][None, :] + _mmT(x[...], Wr1[...])
    h1_o[...] = _leaky(v)
    sdis = sdis2[0] + sdis2[1]
    t = _mmT(sdis, eW1a[...]) * inv + eb1a[...][None, :]
    u = _mmT(t, eW1b[...]) + eb1b[...][None, :]
    dn1m_o[...] = _leaky(u) - c1[...]


def _tc4_body(acch2, accd2, cnt2, h1, c1, Wl2, bl2, Wr2,
              eW2a, eb2a, eW2b, eb2b, Wfc, bfc,
              h2w_o, h2_o, ddn_o):
    cnt = cnt2[0, :, 0:1] + cnt2[1, :, 0:1]
    invm = 1.0 / jnp.maximum(cnt, 1.0)
    inv = 1.0 / (cnt + 1.0)
    acch = acch2[0] + acch2[1]
    v = _mmT(acch * invm, Wl2[...]) + bl2[...][None, :] + _mmT(h1[...], Wr2[...])
    h2 = _leaky(v)
    S = accd2[0] + accd2[1] + cnt * c1[...]
    t = _mmT(S, eW2a[...]) * inv + eb2a[...][None, :]
    u = _mmT(t, eW2b[...]) + eb2b[...][None, :]
    dn2 = _leaky(u)
    wA = Wfc[0, 0:64][None, :]
    wB = Wfc[0, 64:128][None, :]
    h2_o[...] = h2
    h2w_o[...] = h2 * wA
    ddn_o[...] = jnp.sum(dn2 * wB, axis=1, keepdims=True) + bfc[0]


def kernel(x, edge_index, dis, Wl1, bl1, Wr1, Wl2, bl2, Wr2,
           eW1a, eb1a, eW1b, eb1b, eW2a, eb2a, eW2b, eb2b, Wfc, bfc):
    ei = edge_index.astype(jnp.int32)
    rows1 = ei[0]
    cols1 = ei[1]
    rows2d = rows1.reshape(E // CH1, CH1)
    cols2d = cols1.reshape(E // CH1, CH1)
    dis3 = dis.reshape(E // CH1, CH1, DE)
    z128 = jnp.zeros((N, DN), _f32)
    z16 = jnp.zeros((N, DE), _f32)

    c1 = _leaky(eb1a @ eW1b.T + eb1b)[None, :]
    c2 = _leaky(eb2a @ eW2b.T + eb2b)
    c2s = c2 @ Wfc[0, 64:128] + bfc[0]

    accx2, sdis2, cnt2 = _p1(rows2d, cols2d, x, dis3, z128, z16)

    NB = 2000
    grid = N // NB

    def _nblk(shape):
        nd = len(shape)
        blk = shape[:-2] + (NB, shape[-1])
        idx = lambda i: (0,) * (nd - 2) + (i, 0)
        return pl.BlockSpec(blk, idx)

    def _full(shape):
        return pl.BlockSpec(shape, lambda i: (0,) * len(shape))

    h1, dn1m = pl.pallas_call(
        _tc2_body,
        grid=(grid,),
        in_specs=[_nblk((NC, N, DN)), _nblk((NC, N, DE)), _nblk((NC, N, DE)),
                  _nblk((N, DN)), _full(c1.shape), _full(Wl1.shape),
                  _full(bl1.shape), _full(Wr1.shape), _full(eW1a.shape),
                  _full(eb1a.shape), _full(eW1b.shape), _full(eb1b.shape)],
        out_specs=[_nblk((N, HID)), _nblk((N, HID))],
        out_shape=[jax.ShapeDtypeStruct((N, HID), _f32),
                   jax.ShapeDtypeStruct((N, HID), _f32)],
    )(accx2, sdis2, cnt2, x, c1, Wl1, bl1, Wr1, eW1a, eb1a, eW1b, eb1b)

    acch2, accd2 = _p3(rows1, cols1, h1, dn1m, z128)

    h2w, h2, ddn = pl.pallas_call(
        _tc4_body,
        grid=(grid,),
        in_specs=[_nblk((NC, N, HID)), _nblk((NC, N, HID)),
                  _nblk((NC, N, DE)), _nblk((N, HID)), _full(c1.shape),
                  _full(Wl2.shape), _full(bl2.shape), _full(Wr2.shape),
                  _full(eW2a.shape), _full(eb2a.shape), _full(eW2b.shape),
                  _full(eb2b.shape), _full(Wfc.shape), _full(bfc.shape)],
        out_specs=[_nblk((N, 64)), _nblk((N, 64)), _nblk((N, 1))],
        out_shape=[jax.ShapeDtypeStruct((N, 64), _f32),
                   jax.ShapeDtypeStruct((N, 64), _f32),
                   jax.ShapeDtypeStruct((N, 1), _f32)],
    )(acch2, accd2, cnt2, h1, c1, Wl2, bl2, Wr2,
      eW2a, eb2a, eW2b, eb2b, Wfc, bfc)

    dd = jnp.concatenate([ddn[:, 0], jnp.full((E - N,), c2s, _f32)])

    out = _p5(rows1, cols1, h2w, h2, dd)
    return out[:, None]


# speedup vs baseline: 18.2492x; 1.0017x over previous
import functools

import jax
import jax.numpy as jnp
from jax import lax
from jax.experimental import pallas as pl
from jax.experimental.pallas import tpu as pltpu
from jax.experimental.pallas import tpu_sc as plsc

N = 10000
E = 320000
DN = 128
DE = 16
HID = 128
NC = 2
NS = 16
NW = NC * NS
NPT = 624

CH1 = 80
NCH1 = (E // NW) // CH1
CH = 128
NCHF = 78
SPT = NCHF * CH
EXTRA0 = NW * SPT
NEX = (E - EXTRA0) // CH
NLO = N // CH
LOTAIL = N - NLO * CH

_mesh = plsc.VectorSubcoreMesh(core_axis_name="c", subcore_axis_name="s")
_f32 = jnp.float32


def _fill_ones(ref, rows):
    def body(r, _):
        ref[r, pl.ds(0, 16)] = jnp.ones((16,), _f32)
        return 0
    lax.fori_loop(0, rows, body, 0)


def _tile_rows(s, fn):
    fn(s * NPT, NPT)

    @pl.when(s == NS - 1)
    def _():
        fn(NS * NPT, N - NS * NPT)


@functools.partial(
    pl.kernel, mesh=_mesh,
    compiler_params=pltpu.CompilerParams(use_tc_tiling_on_sc=False),
    out_type=[
        jax.ShapeDtypeStruct((NC, N, DN), _f32),
        jax.ShapeDtypeStruct((NC, N, DE), _f32),
        jax.ShapeDtypeStruct((NC, N, DE), _f32),
    ],
    scratch_types=[
        pltpu.VMEM((3, CH1), jnp.int32),
        pltpu.VMEM((3, CH1), jnp.int32),
        pltpu.VMEM((2, CH1, DN), _f32),
        pltpu.VMEM((2, CH1, DE), _f32),
        pltpu.VMEM((CH1, DE), _f32),
        pltpu.VMEM_SHARED((N, DN), _f32),
        pltpu.VMEM_SHARED((N, DE), _f32),
        pltpu.VMEM_SHARED((N, DE), _f32),
        pltpu.SemaphoreType.DMA,
        pltpu.SemaphoreType.DMA,
        pltpu.SemaphoreType.DMA,
    ],
)
def _p1(rows2d, cols2d, x, dis3, z128, z16,
        accx_o, sdis_o, cnt_o,
        rowb, colb, xrows, disb, oneb, accx_s, sdis_s, cnt_s,
        sem_l, sem_g, sem_s):
    c = lax.axis_index("c")
    s = lax.axis_index("s")
    wid = s * NC + c
    base = wid * NCH1
    _fill_ones(oneb, CH1)

    def init(o, n):
        pltpu.sync_copy(z128.at[pl.ds(o, n)], accx_s.at[pl.ds(o, n)])
        pltpu.sync_copy(z16.at[pl.ds(o, n)], sdis_s.at[pl.ds(o, n)])
        pltpu.sync_copy(z16.at[pl.ds(o, n)], cnt_s.at[pl.ds(o, n)])

    _tile_rows(s, init)
    plsc.subcore_barrier()

    def ld(slot, cid):
        pltpu.async_copy(rows2d.at[cid], rowb.at[slot], sem_l)
        pltpu.async_copy(cols2d.at[cid], colb.at[slot], sem_l)

    def ldw(slot):
        pltpu.make_async_copy(rows2d.at[0], rowb.at[slot], sem_l).wait()
        pltpu.make_async_copy(cols2d.at[0], colb.at[slot], sem_l).wait()

    def sc_start(islot, dslot):
        pltpu.async_copy(xrows.at[dslot], accx_s.at[colb.at[islot]], sem_s,
                         add=True)
        pltpu.async_copy(disb.at[dslot], sdis_s.at[colb.at[islot]], sem_s,
                         add=True)
        pltpu.async_copy(oneb, cnt_s.at[colb.at[islot]], sem_s, add=True)

    def sc_wait(islot, dslot):
        pltpu.make_async_copy(xrows.at[dslot], accx_s.at[colb.at[islot]],
                              sem_s).wait()
        pltpu.make_async_copy(disb.at[dslot], sdis_s.at[colb.at[islot]],
                              sem_s).wait()
        pltpu.make_async_copy(oneb, cnt_s.at[colb.at[islot]], sem_s).wait()

    def gst(islot, dslot, cid):
        pltpu.async_copy(x.at[rowb.at[islot]], xrows.at[dslot], sem_g)
        pltpu.async_copy(dis3.at[cid], disb.at[dslot], sem_g)

    def gwt(islot, dslot):
        pltpu.make_async_copy(x.at[rowb.at[islot]], xrows.at[dslot],
                              sem_g).wait()
        pltpu.make_async_copy(dis3.at[0], disb.at[dslot], sem_g).wait()

    ld(0, base)
    ldw(0)
    gst(0, 0, base)
    ld(1, base + 1)

    def body(j, _):
        sj = lax.rem(j, 3)
        sn = lax.rem(j + 1, 3)
        sp = lax.rem(j + 2, 3)
        dj = lax.rem(j, 2)
        dn = 1 - dj

        @pl.when(j > 0)
        def _():
            sc_wait(sp, dn)

        gwt(sj, dj)

        @pl.when(j + 1 < NCH1)
        def _():
            ldw(sn)
            gst(sn, dn, base + j + 1)

        @pl.when(j + 2 < NCH1)
        def _():
            ld(sp, base + j + 2)

        sc_start(sj, dj)
        return 0

    lax.fori_loop(0, NCH1, body, 0)
    sc_wait(lax.rem(NCH1 - 1, 3), lax.rem(NCH1 - 1, 2))
    plsc.subcore_barrier()

    def writeout(o, n):
        pltpu.sync_copy(accx_s.at[pl.ds(o, n)], accx_o.at[c, pl.ds(o, n)])
        pltpu.sync_copy(sdis_s.at[pl.ds(o, n)], sdis_o.at[c, pl.ds(o, n)])
        pltpu.sync_copy(cnt_s.at[pl.ds(o, n)], cnt_o.at[c, pl.ds(o, n)])

    _tile_rows(s, writeout)


@functools.partial(
    pl.kernel, mesh=_mesh,
    compiler_params=pltpu.CompilerParams(use_tc_tiling_on_sc=False),
    out_type=[
        jax.ShapeDtypeStruct((NC, N, HID), _f32),
        jax.ShapeDtypeStruct((NC, N, HID), _f32),
    ],
    scratch_types=[
        pltpu.VMEM((3, CH), jnp.int32),
        pltpu.VMEM((3, CH), jnp.int32),
        pltpu.VMEM((2, CH, HID), _f32),
        pltpu.VMEM((LOTAIL,), jnp.int32),
        pltpu.VMEM((LOTAIL, HID), _f32),
        pltpu.VMEM_SHARED((N, HID), _f32),
        pltpu.SemaphoreType.DMA,
        pltpu.SemaphoreType.DMA,
        pltpu.SemaphoreType.DMA,
    ],
)
def _p3(rows1, cols1, h1, dn1m, z128, acch_o, accd_o,
        rowb, colb, hrows, colb16, drows16, acc_s, sem_l, sem_g, sem_s):
    c = lax.axis_index("c")
    s = lax.axis_index("s")
    wid = s * NC + c
    ebase = wid * SPT

    def zero_table(o, n):
        pltpu.sync_copy(z128.at[pl.ds(o, n)], acc_s.at[pl.ds(o, n)])

    _tile_rows(s, zero_table)
    plsc.subcore_barrier()

    def ld(slot, st):
        pltpu.async_copy(rows1.at[pl.ds(st, CH)], rowb.at[slot], sem_l)
        pltpu.async_copy(cols1.at[pl.ds(st, CH)], colb.at[slot], sem_l)

    def ldw(slot):
        pltpu.make_async_copy(rows1.at[pl.ds(0, CH)], rowb.at[slot],
                              sem_l).wait()
        pltpu.make_async_copy(cols1.at[pl.ds(0, CH)], colb.at[slot],
                              sem_l).wait()

    ld(0, ebase)
    ldw(0)
    pltpu.async_copy(h1.at[rowb.at[0]], hrows.at[0], sem_g)
    ld(1, ebase + CH)

    def body(j, _):
        sj = lax.rem(j, 3)
        sn = lax.rem(j + 1, 3)
        sp = lax.rem(j + 2, 3)
        dj = lax.rem(j, 2)
        dn = 1 - dj

        @pl.when(j > 0)
        def _():
            pltpu.make_async_copy(hrows.at[dn], acc_s.at[colb.at[sp]],
                                  sem_s).wait()

        pltpu.make_async_copy(h1.at[rowb.at[sj]], hrows.at[dj], sem_g).wait()

        @pl.when(j + 1 < NCHF)
        def _():
            ldw(sn)
            pltpu.async_copy(h1.at[rowb.at[sn]], hrows.at[dn], sem_g)

        @pl.when(j + 2 < NCHF)
        def _():
            ld(sp, ebase + (j + 2) * CH)

        pltpu.async_copy(hrows.at[dj], acc_s.at[colb.at[sj]], sem_s,
                         add=True)
        return 0

    lax.fori_loop(0, NCHF, body, 0)
    pltpu.make_async_copy(hrows.at[lax.rem(NCHF - 1, 2)],
                          acc_s.at[colb.at[lax.rem(NCHF - 1, 3)]],
                          sem_s).wait()

    @pl.when(wid < NEX)
    def _():
        st = EXTRA0 + wid * CH
        ld(0, st)
        ldw(0)
        pltpu.async_copy(h1.at[rowb.at[0]], hrows.at[0], sem_g)
        pltpu.make_async_copy(h1.at[rowb.at[0]], hrows.at[0], sem_g).wait()
        pltpu.async_copy(hrows.at[0], acc_s.at[colb.at[0]], sem_s, add=True)
        pltpu.make_async_copy(hrows.at[0], acc_s.at[colb.at[0]],
                              sem_s).wait()

    plsc.subcore_barrier()

    def wr_acch(o, n):
        pltpu.sync_copy(acc_s.at[pl.ds(o, n)], acch_o.at[c, pl.ds(o, n)])
        pltpu.sync_copy(z128.at[pl.ds(o, n)], acc_s.at[pl.ds(o, n)])

    _tile_rows(s, wr_acch)
    plsc.subcore_barrier()

    for t in range((NLO + NW - 1) // NW):
        cid = wid + NW * t

        @pl.when(cid < NLO)
        def _():
            st = cid * CH
            pltpu.sync_copy(cols1.at[pl.ds(st, CH)], colb.at[0])
            pltpu.sync_copy(dn1m.at[pl.ds(st, CH)], hrows.at[0])
            pltpu.async_copy(hrows.at[0], acc_s.at[colb.at[0]], sem_s,
                             add=True)
            pltpu.make_async_copy(hrows.at[0], acc_s.at[colb.at[0]],
                                  sem_s).wait()

    @pl.when(wid == NW - 1)
    def _():
        st = NLO * CH
        pltpu.sync_copy(cols1.at[pl.ds(st, LOTAIL)], colb16)
        pltpu.sync_copy(dn1m.at[pl.ds(st, LOTAIL)], drows16)
        pltpu.async_copy(drows16, acc_s.at[colb16], sem_s, add=True)
        pltpu.make_async_copy(drows16, acc_s.at[colb16], sem_s).wait()

    plsc.subcore_barrier()
    _tile_rows(s, lambda o, n: pltpu.sync_copy(
        acc_s.at[pl.ds(o, n)], accd_o.at[c, pl.ds(o, n)]))


@functools.partial(
    pl.kernel, mesh=_mesh,
    out_type=jax.ShapeDtypeStruct((E,), _f32),
    compiler_params=pltpu.CompilerParams(use_tc_tiling_on_sc=False),
    scratch_types=[
        pltpu.VMEM((3, CH), jnp.int32),
        pltpu.VMEM((3, CH), jnp.int32),
        pltpu.VMEM((3, CH, 64), _f32),
        pltpu.VMEM((3, CH, 64), _f32),
        pltpu.VMEM((3, CH), _f32),
        pltpu.VMEM((3, CH), _f32),
        pltpu.VMEM((16,), _f32),
        pltpu.SemaphoreType.DMA,
        pltpu.SemaphoreType.DMA,
        pltpu.SemaphoreType.DMA,
    ],
)
def _p5(rows1, cols1, h2w, h2, ddp, c2v, out,
        rowb, colb, rbuf, cbuf, ddb, ob, c2b, sem_l, sem_g, sem_o):
    c = lax.axis_index("c")
    s = lax.axis_index("s")
    wid = s * NC + c
    ebase = wid * SPT
    pltpu.sync_copy(c2v, c2b)
    vc2 = c2b[pl.ds(0, 16)]

    def ld(slot, st):
        pltpu.async_copy(rows1.at[pl.ds(st, CH)], rowb.at[slot], sem_l)
        pltpu.async_copy(cols1.at[pl.ds(st, CH)], colb.at[slot], sem_l)

        @pl.when(st < N)
        def _():
            pltpu.async_copy(ddp.at[pl.ds(st, CH)], ddb.at[slot], sem_l)

    def ldw(slot, st):
        pltpu.make_async_copy(rows1.at[pl.ds(0, CH)], rowb.at[slot],
                              sem_l).wait()
        pltpu.make_async_copy(cols1.at[pl.ds(0, CH)], colb.at[slot],
                              sem_l).wait()

        @pl.when(st < N)
        def _():
            pltpu.make_async_copy(ddp.at[pl.ds(0, CH)], ddb.at[slot],
                                  sem_l).wait()

    def gst(slot):
        pltpu.async_copy(h2w.at[rowb.at[slot]], rbuf.at[slot], sem_g)
        pltpu.async_copy(h2.at[colb.at[slot]], cbuf.at[slot], sem_g)

    def gw(slot):
        pltpu.make_async_copy(h2w.at[rowb.at[slot]], rbuf.at[slot],
                              sem_g).wait()
        pltpu.make_async_copy(h2.at[colb.at[slot]], cbuf.at[slot],
                              sem_g).wait()

    lane = lax.iota(jnp.int32, 16)
    p8 = lane ^ 8
    p4 = lane ^ 4
    p2 = lane ^ 2
    p1 = lane ^ 1

    def compute(sj, st):
        def group(g, _):
            def edot(e16, accv):
                e = g * 16 + e16
                v = rbuf[sj, e, pl.ds(0, 16)] * cbuf[sj, e, pl.ds(0, 16)]
                v = v + rbuf[sj, e, pl.ds(16, 16)] * cbuf[sj, e, pl.ds(16, 16)]
                v = v + rbuf[sj, e, pl.ds(32, 16)] * cbuf[sj, e, pl.ds(32, 16)]
                v = v + rbuf[sj, e, pl.ds(48, 16)] * cbuf[sj, e, pl.ds(48, 16)]
                v = v + v[p8]
                v = v + v[p4]
                v = v + v[p2]
                v = v + v[p1]
                return jnp.where(lane == e16, v, accv)

            accv = lax.fori_loop(0, 16, edot, jnp.zeros((16,), _f32))
            eid = st + g * 16 + lane
            ddv = jnp.where(eid < N, ddb[sj, pl.ds(g * 16, 16)], vc2)
            ob[sj, pl.ds(g * 16, 16)] = accv + ddv
            return 0

        lax.fori_loop(0, CH // 16, group, 0)

    ld(0, ebase)
    ldw(0, ebase)
    gst(0)
    ld(1, ebase + CH)

    def body(j, _):
        sj = lax.rem(j, 3)
        sn = lax.rem(j + 1, 3)
        sp = lax.rem(j + 2, 3)

        @pl.when(j > 0)
        def _():
            pltpu.make_async_copy(
                ob.at[sp], out.at[pl.ds(ebase + (j - 1) * CH, CH)],
                sem_o).wait()

        gw(sj)

        @pl.when(j + 1 < NCHF)
        def _():
            ldw(sn, ebase + (j + 1) * CH)
            gst(sn)

        @pl.when(j + 2 < NCHF)
        def _():
            ld(sp, ebase + (j + 2) * CH)

        compute(sj, ebase + j * CH)
        pltpu.async_copy(ob.at[sj], out.at[pl.ds(ebase + j * CH, CH)], sem_o)
        return 0

    lax.fori_loop(0, NCHF, body, 0)
    sl = lax.rem(NCHF - 1, 3)
    pltpu.make_async_copy(ob.at[sl],
                          out.at[pl.ds(ebase + (NCHF - 1) * CH, CH)],
                          sem_o).wait()

    @pl.when(wid < NEX)
    def _():
        st = EXTRA0 + wid * CH
        ld(0, st)
        ldw(0, st)
        gst(0)
        gw(0)
        compute(0, st)
        pltpu.async_copy(ob.at[0], out.at[pl.ds(st, CH)], sem_o)
        pltpu.make_async_copy(ob.at[0], out.at[pl.ds(st, CH)], sem_o).wait()


def _mmT(a, w):
    return lax.dot_general(a, w, (((1,), (1,)), ((), ())),
                           preferred_element_type=_f32)


def _leaky(v):
    return jnp.where(v >= 0, v, 0.01 * v)


def _tc2_body(accx2, sdis2, cnt2, x, c1, Wl1, bl1, Wr1, eW1a, eb1a, eW1b,
              eb1b, h1_o, dn1m_o):
    cnt = cnt2[0, :, 0:1] + cnt2[1, :, 0:1]
    accx = accx2[0] + accx2[1]
    invm = 1.0 / jnp.maximum(cnt, 1.0)
    inv = 1.0 / (cnt + 1.0)
    v = _mmT(accx * invm, Wl1[...]) + bl1[...][None, :] + _mmT(x[...], Wr1[...])
    h1_o[...] = _leaky(v)
    sdis = sdis2[0] + sdis2[1]
    t = _mmT(sdis, eW1a[...]) * inv + eb1a[...][None, :]
    u = _mmT(t, eW1b[...]) + eb1b[...][None, :]
    dn1m_o[...] = _leaky(u) - c1[...]


def _tc4_body(acch2, accd2, cnt2, h1, c1, Wl2, bl2, Wr2,
              eW2a, eb2a, eW2b, eb2b, Wfc, bfc,
              h2w_o, h2_o, ddn_o):
    cnt = cnt2[0, :, 0:1] + cnt2[1, :, 0:1]
    invm = 1.0 / jnp.maximum(cnt, 1.0)
    inv = 1.0 / (cnt + 1.0)
    acch = acch2[0] + acch2[1]
    v = _mmT(acch * invm, Wl2[...]) + bl2[...][None, :] + _mmT(h1[...], Wr2[...])
    h2 = _leaky(v)
    S = accd2[0] + accd2[1] + cnt * c1[...]
    t = _mmT(S, eW2a[...]) * inv + eb2a[...][None, :]
    u = _mmT(t, eW2b[...]) + eb2b[...][None, :]
    dn2 = _leaky(u)
    wA = Wfc[0, 0:64][None, :]
    wB = Wfc[0, 64:128][None, :]
    h2_o[...] = h2
    h2w_o[...] = h2 * wA
    ddn_o[...] = jnp.sum(dn2 * wB, axis=1, keepdims=True) + bfc[0]


def kernel(x, edge_index, dis, Wl1, bl1, Wr1, Wl2, bl2, Wr2,
           eW1a, eb1a, eW1b, eb1b, eW2a, eb2a, eW2b, eb2b, Wfc, bfc):
    ei = edge_index.astype(jnp.int32)
    rows1 = ei[0]
    cols1 = ei[1]
    rows2d = rows1.reshape(E // CH1, CH1)
    cols2d = cols1.reshape(E // CH1, CH1)
    dis3 = dis.reshape(E // CH1, CH1, DE)
    z128 = jnp.zeros((N, DN), _f32)
    z16 = jnp.zeros((N, DE), _f32)

    c1 = _leaky(eb1a @ eW1b.T + eb1b)[None, :]
    c2 = _leaky(eb2a @ eW2b.T + eb2b)
    c2s = c2 @ Wfc[0, 64:128] + bfc[0]

    accx2, sdis2, cnt2 = _p1(rows2d, cols2d, x, dis3, z128, z16)

    NB = 2000
    grid = N // NB

    def _nblk(shape):
        nd = len(shape)
        blk = shape[:-2] + (NB, shape[-1])
        idx = lambda i: (0,) * (nd - 2) + (i, 0)
        return pl.BlockSpec(blk, idx)

    def _full(shape):
        return pl.BlockSpec(shape, lambda i: (0,) * len(shape))

    h1, dn1m = pl.pallas_call(
        _tc2_body,
        grid=(grid,),
        in_specs=[_nblk((NC, N, DN)), _nblk((NC, N, DE)), _nblk((NC, N, DE)),
                  _nblk((N, DN)), _full(c1.shape), _full(Wl1.shape),
                  _full(bl1.shape), _full(Wr1.shape), _full(eW1a.shape),
                  _full(eb1a.shape), _full(eW1b.shape), _full(eb1b.shape)],
        out_specs=[_nblk((N, HID)), _nblk((N, HID))],
        out_shape=[jax.ShapeDtypeStruct((N, HID), _f32),
                   jax.ShapeDtypeStruct((N, HID), _f32)],
    )(accx2, sdis2, cnt2, x, c1, Wl1, bl1, Wr1, eW1a, eb1a, eW1b, eb1b)

    acch2, accd2 = _p3(rows1, cols1, h1, dn1m, z128)

    h2w, h2, ddn = pl.pallas_call(
        _tc4_body,
        grid=(grid,),
        in_specs=[_nblk((NC, N, HID)), _nblk((NC, N, HID)),
                  _nblk((NC, N, DE)), _nblk((N, HID)), _full(c1.shape),
                  _full(Wl2.shape), _full(bl2.shape), _full(Wr2.shape),
                  _full(eW2a.shape), _full(eb2a.shape), _full(eW2b.shape),
                  _full(eb2b.shape), _full(Wfc.shape), _full(bfc.shape)],
        out_specs=[_nblk((N, 64)), _nblk((N, 64)),
                   pl.BlockSpec((NB, 1), lambda i: (i, 0))],
        out_shape=[jax.ShapeDtypeStruct((N, 64), _f32),
                   jax.ShapeDtypeStruct((N, 64), _f32),
                   jax.ShapeDtypeStruct(((NLO + 1) * CH, 1), _f32)],
    )(acch2, accd2, cnt2, h1, c1, Wl2, bl2, Wr2,
      eW2a, eb2a, eW2b, eb2b, Wfc, bfc)

    c2v = jnp.full((16,), c2s, _f32)
    out = _p5(rows1, cols1, h2w, h2, ddn[:, 0], c2v)
    return out[:, None]


# speedup vs baseline: 18.2704x; 1.0012x over previous
import functools

import jax
import jax.numpy as jnp
from jax import lax
from jax.experimental import pallas as pl
from jax.experimental.pallas import tpu as pltpu
from jax.experimental.pallas import tpu_sc as plsc

N = 10000
E = 320000
DN = 128
DE = 16
HID = 128
NC = 2
NS = 16
NW = NC * NS
NPT = 624

CH1 = 80
NCH1 = (E // NW) // CH1
CH = 128
NCHF = 78
SPT = NCHF * CH
EXTRA0 = NW * SPT
NEX = (E - EXTRA0) // CH
NLO = N // CH
LOTAIL = N - NLO * CH

_mesh = plsc.VectorSubcoreMesh(core_axis_name="c", subcore_axis_name="s")
_f32 = jnp.float32


def _fill_ones(ref, rows):
    def body(r, _):
        ref[r, pl.ds(0, 16)] = jnp.ones((16,), _f32)
        return 0
    lax.fori_loop(0, rows, body, 0)


def _tile_rows(s, fn):
    fn(s * NPT, NPT)

    @pl.when(s == NS - 1)
    def _():
        fn(NS * NPT, N - NS * NPT)


@functools.partial(
    pl.kernel, mesh=_mesh,
    compiler_params=pltpu.CompilerParams(use_tc_tiling_on_sc=False),
    out_type=[
        jax.ShapeDtypeStruct((NC, N, DN), _f32),
        jax.ShapeDtypeStruct((NC, N, DE), _f32),
        jax.ShapeDtypeStruct((NC, N, DE), _f32),
    ],
    scratch_types=[
        pltpu.VMEM((3, CH1), jnp.int32),
        pltpu.VMEM((3, CH1), jnp.int32),
        pltpu.VMEM((2, CH1, DN), _f32),
        pltpu.VMEM((2, CH1, DE), _f32),
        pltpu.VMEM((CH1, DE), _f32),
        pltpu.VMEM_SHARED((N, DN), _f32),
        pltpu.VMEM_SHARED((N, DE), _f32),
        pltpu.VMEM_SHARED((N, DE), _f32),
        pltpu.SemaphoreType.DMA,
        pltpu.SemaphoreType.DMA,
        pltpu.SemaphoreType.DMA,
    ],
)
def _p1(ei, x, dis, z128, z16,
        accx_o, sdis_o, cnt_o,
        rowb, colb, xrows, disb, oneb, accx_s, sdis_s, cnt_s,
        sem_l, sem_g, sem_s):
    c = lax.axis_index("c")
    s = lax.axis_index("s")
    wid = s * NC + c
    base = wid * NCH1 * CH1
    _fill_ones(oneb, CH1)

    def init(o, n):
        pltpu.sync_copy(z128.at[pl.ds(o, n)], accx_s.at[pl.ds(o, n)])
        pltpu.sync_copy(z16.at[pl.ds(o, n)], sdis_s.at[pl.ds(o, n)])
        pltpu.sync_copy(z16.at[pl.ds(o, n)], cnt_s.at[pl.ds(o, n)])

    _tile_rows(s, init)
    plsc.subcore_barrier()

    def ld(slot, st):
        pltpu.async_copy(ei.at[0, pl.ds(st, CH1)], rowb.at[slot], sem_l)
        pltpu.async_copy(ei.at[1, pl.ds(st, CH1)], colb.at[slot], sem_l)

    def ldw(slot):
        pltpu.make_async_copy(ei.at[0, pl.ds(0, CH1)], rowb.at[slot],
                              sem_l).wait()
        pltpu.make_async_copy(ei.at[1, pl.ds(0, CH1)], colb.at[slot],
                              sem_l).wait()

    def sc_start(islot, dslot):
        pltpu.async_copy(xrows.at[dslot], accx_s.at[colb.at[islot]], sem_s,
                         add=True)
        pltpu.async_copy(disb.at[dslot], sdis_s.at[colb.at[islot]], sem_s,
                         add=True)
        pltpu.async_copy(oneb, cnt_s.at[colb.at[islot]], sem_s, add=True)

    def sc_wait(islot, dslot):
        pltpu.make_async_copy(xrows.at[dslot], accx_s.at[colb.at[islot]],
                              sem_s).wait()
        pltpu.make_async_copy(disb.at[dslot], sdis_s.at[colb.at[islot]],
                              sem_s).wait()
        pltpu.make_async_copy(oneb, cnt_s.at[colb.at[islot]], sem_s).wait()

    def gst(islot, dslot, st):
        pltpu.async_copy(x.at[rowb.at[islot]], xrows.at[dslot], sem_g)
        pltpu.async_copy(dis.at[pl.ds(st, CH1)], disb.at[dslot], sem_g)

    def gwt(islot, dslot):
        pltpu.make_async_copy(x.at[rowb.at[islot]], xrows.at[dslot],
                              sem_g).wait()
        pltpu.make_async_copy(dis.at[pl.ds(0, CH1)], disb.at[dslot],
                              sem_g).wait()

    ld(0, base)
    ldw(0)
    gst(0, 0, base)
    ld(1, base + CH1)

    def body(j, _):
        sj = lax.rem(j, 3)
        sn = lax.rem(j + 1, 3)
        sp = lax.rem(j + 2, 3)
        dj = lax.rem(j, 2)
        dn = 1 - dj

        @pl.when(j > 0)
        def _():
            sc_wait(sp, dn)

        gwt(sj, dj)

        @pl.when(j + 1 < NCH1)
        def _():
            ldw(sn)
            gst(sn, dn, base + (j + 1) * CH1)

        @pl.when(j + 2 < NCH1)
        def _():
            ld(sp, base + (j + 2) * CH1)

        sc_start(sj, dj)
        return 0

    lax.fori_loop(0, NCH1, body, 0)
    sc_wait(lax.rem(NCH1 - 1, 3), lax.rem(NCH1 - 1, 2))
    plsc.subcore_barrier()

    def writeout(o, n):
        pltpu.sync_copy(accx_s.at[pl.ds(o, n)], accx_o.at[c, pl.ds(o, n)])
        pltpu.sync_copy(sdis_s.at[pl.ds(o, n)], sdis_o.at[c, pl.ds(o, n)])
        pltpu.sync_copy(cnt_s.at[pl.ds(o, n)], cnt_o.at[c, pl.ds(o, n)])

    _tile_rows(s, writeout)


@functools.partial(
    pl.kernel, mesh=_mesh,
    compiler_params=pltpu.CompilerParams(use_tc_tiling_on_sc=False),
    out_type=[
        jax.ShapeDtypeStruct((NC, N, HID), _f32),
        jax.ShapeDtypeStruct((NC, N, HID), _f32),
    ],
    scratch_types=[
        pltpu.VMEM((3, CH), jnp.int32),
        pltpu.VMEM((3, CH), jnp.int32),
        pltpu.VMEM((2, CH, HID), _f32),
        pltpu.VMEM((LOTAIL,), jnp.int32),
        pltpu.VMEM((LOTAIL, HID), _f32),
        pltpu.VMEM_SHARED((N, HID), _f32),
        pltpu.SemaphoreType.DMA,
        pltpu.SemaphoreType.DMA,
        pltpu.SemaphoreType.DMA,
    ],
)
def _p3(ei, h1, dn1m, z128, acch_o, accd_o,
        rowb, colb, hrows, colb16, drows16, acc_s, sem_l, sem_g, sem_s):
    c = lax.axis_index("c")
    s = lax.axis_index("s")
    wid = s * NC + c
    ebase = wid * SPT

    def zero_table(o, n):
        pltpu.sync_copy(z128.at[pl.ds(o, n)], acc_s.at[pl.ds(o, n)])

    _tile_rows(s, zero_table)
    plsc.subcore_barrier()

    def ld(slot, st):
        pltpu.async_copy(ei.at[0, pl.ds(st, CH)], rowb.at[slot], sem_l)
        pltpu.async_copy(ei.at[1, pl.ds(st, CH)], colb.at[slot], sem_l)

    def ldw(slot):
        pltpu.make_async_copy(ei.at[0, pl.ds(0, CH)], rowb.at[slot],
                              sem_l).wait()
        pltpu.make_async_copy(ei.at[1, pl.ds(0, CH)], colb.at[slot],
                              sem_l).wait()

    ld(0, ebase)
    ldw(0)
    pltpu.async_copy(h1.at[rowb.at[0]], hrows.at[0], sem_g)
    ld(1, ebase + CH)

    def body(j, _):
        sj = lax.rem(j, 3)
        sn = lax.rem(j + 1, 3)
        sp = lax.rem(j + 2, 3)
        dj = lax.rem(j, 2)
        dn = 1 - dj

        @pl.when(j > 0)
        def _():
            pltpu.make_async_copy(hrows.at[dn], acc_s.at[colb.at[sp]],
                                  sem_s).wait()

        pltpu.make_async_copy(h1.at[rowb.at[sj]], hrows.at[dj], sem_g).wait()

        @pl.when(j + 1 < NCHF)
        def _():
            ldw(sn)
            pltpu.async_copy(h1.at[rowb.at[sn]], hrows.at[dn], sem_g)

        @pl.when(j + 2 < NCHF)
        def _():
            ld(sp, ebase + (j + 2) * CH)

        pltpu.async_copy(hrows.at[dj], acc_s.at[colb.at[sj]], sem_s,
                         add=True)
        return 0

    lax.fori_loop(0, NCHF, body, 0)
    pltpu.make_async_copy(hrows.at[lax.rem(NCHF - 1, 2)],
                          acc_s.at[colb.at[lax.rem(NCHF - 1, 3)]],
                          sem_s).wait()

    @pl.when(wid < NEX)
    def _():
        st = EXTRA0 + wid * CH
        ld(0, st)
        ldw(0)
        pltpu.async_copy(h1.at[rowb.at[0]], hrows.at[0], sem_g)
        pltpu.make_async_copy(h1.at[rowb.at[0]], hrows.at[0], sem_g).wait()
        pltpu.async_copy(hrows.at[0], acc_s.at[colb.at[0]], sem_s, add=True)
        pltpu.make_async_copy(hrows.at[0], acc_s.at[colb.at[0]],
                              sem_s).wait()

    plsc.subcore_barrier()

    def wr_acch(o, n):
        pltpu.sync_copy(acc_s.at[pl.ds(o, n)], acch_o.at[c, pl.ds(o, n)])
        pltpu.sync_copy(z128.at[pl.ds(o, n)], acc_s.at[pl.ds(o, n)])

    _tile_rows(s, wr_acch)
    plsc.subcore_barrier()

    for t in range((NLO + NW - 1) // NW):
        cid = wid + NW * t

        @pl.when(cid < NLO)
        def _():
            st = cid * CH
            pltpu.sync_copy(ei.at[1, pl.ds(st, CH)], colb.at[0])
            pltpu.sync_copy(dn1m.at[pl.ds(st, CH)], hrows.at[0])
            pltpu.async_copy(hrows.at[0], acc_s.at[colb.at[0]], sem_s,
                             add=True)
            pltpu.make_async_copy(hrows.at[0], acc_s.at[colb.at[0]],
                                  sem_s).wait()

    @pl.when(wid == NW - 1)
    def _():
        st = NLO * CH
        pltpu.sync_copy(ei.at[1, pl.ds(st, LOTAIL)], colb16)
        pltpu.sync_copy(dn1m.at[pl.ds(st, LOTAIL)], drows16)
        pltpu.async_copy(drows16, acc_s.at[colb16], sem_s, add=True)
        pltpu.make_async_copy(drows16, acc_s.at[colb16], sem_s).wait()

    plsc.subcore_barrier()
    _tile_rows(s, lambda o, n: pltpu.sync_copy(
        acc_s.at[pl.ds(o, n)], accd_o.at[c, pl.ds(o, n)]))


@functools.partial(
    pl.kernel, mesh=_mesh,
    out_type=jax.ShapeDtypeStruct((E,), _f32),
    compiler_params=pltpu.CompilerParams(use_tc_tiling_on_sc=False),
    scratch_types=[
        pltpu.VMEM((3, CH), jnp.int32),
        pltpu.VMEM((3, CH), jnp.int32),
        pltpu.VMEM((3, CH, 64), _f32),
        pltpu.VMEM((3, CH, 64), _f32),
        pltpu.VMEM((3, CH), _f32),
        pltpu.VMEM((3, CH), _f32),
        pltpu.VMEM((16,), _f32),
        pltpu.SemaphoreType.DMA,
        pltpu.SemaphoreType.DMA,
        pltpu.SemaphoreType.DMA,
    ],
)
def _p5(ei, h2w, h2, ddp, c2v, out,
        rowb, colb, rbuf, cbuf, ddb, ob, c2b, sem_l, sem_g, sem_o):
    c = lax.axis_index("c")
    s = lax.axis_index("s")
    wid = s * NC + c
    ebase = wid * SPT
    pltpu.sync_copy(c2v, c2b)
    vc2 = c2b[pl.ds(0, 16)]

    def ld(slot, st):
        pltpu.async_copy(ei.at[0, pl.ds(st, CH)], rowb.at[slot], sem_l)
        pltpu.async_copy(ei.at[1, pl.ds(st, CH)], colb.at[slot], sem_l)

        @pl.when(st < N)
        def _():
            pltpu.async_copy(ddp.at[pl.ds(st, CH)], ddb.at[slot], sem_l)

    def ldw(slot, st):
        pltpu.make_async_copy(ei.at[0, pl.ds(0, CH)], rowb.at[slot],
                              sem_l).wait()
        pltpu.make_async_copy(ei.at[1, pl.ds(0, CH)], colb.at[slot],
                              sem_l).wait()

        @pl.when(st < N)
        def _():
            pltpu.make_async_copy(ddp.at[pl.ds(0, CH)], ddb.at[slot],
                                  sem_l).wait()

    def gst(slot):
        pltpu.async_copy(h2w.at[rowb.at[slot]], rbuf.at[slot], sem_g)
        pltpu.async_copy(h2.at[colb.at[slot]], cbuf.at[slot], sem_g)

    def gw(slot):
        pltpu.make_async_copy(h2w.at[rowb.at[slot]], rbuf.at[slot],
                              sem_g).wait()
        pltpu.make_async_copy(h2.at[colb.at[slot]], cbuf.at[slot],
                              sem_g).wait()

    lane = lax.iota(jnp.int32, 16)
    p8 = lane ^ 8
    p4 = lane ^ 4
    p2 = lane ^ 2
    p1 = lane ^ 1

    def compute(sj, st):
        def group(g, _):
            def edot(e16, accv):
                e = g * 16 + e16
                v = rbuf[sj, e, pl.ds(0, 16)] * cbuf[sj, e, pl.ds(0, 16)]
                v = v + rbuf[sj, e, pl.ds(16, 16)] * cbuf[sj, e, pl.ds(16, 16)]
                v = v + rbuf[sj, e, pl.ds(32, 16)] * cbuf[sj, e, pl.ds(32, 16)]
                v = v + rbuf[sj, e, pl.ds(48, 16)] * cbuf[sj, e, pl.ds(48, 16)]
                v = v + v[p8]
                v = v + v[p4]
                v = v + v[p2]
                v = v + v[p1]
                return jnp.where(lane == e16, v, accv)

            accv = lax.fori_loop(0, 16, edot, jnp.zeros((16,), _f32))
            eid = st + g * 16 + lane
            ddv = jnp.where(eid < N, ddb[sj, pl.ds(g * 16, 16)], vc2)
            ob[sj, pl.ds(g * 16, 16)] = accv + ddv
            return 0

        lax.fori_loop(0, CH // 16, group, 0)

    ld(0, ebase)
    ldw(0, ebase)
    gst(0)
    ld(1, ebase + CH)

    def body(j, _):
        sj = lax.rem(j, 3)
        sn = lax.rem(j + 1, 3)
        sp = lax.rem(j + 2, 3)

        @pl.when(j > 0)
        def _():
            pltpu.make_async_copy(
                ob.at[sp], out.at[pl.ds(ebase + (j - 1) * CH, CH)],
                sem_o).wait()

        gw(sj)

        @pl.when(j + 1 < NCHF)
        def _():
            ldw(sn, ebase + (j + 1) * CH)
            gst(sn)

        @pl.when(j + 2 < NCHF)
        def _():
            ld(sp, ebase + (j + 2) * CH)

        compute(sj, ebase + j * CH)
        pltpu.async_copy(ob.at[sj], out.at[pl.ds(ebase + j * CH, CH)], sem_o)
        return 0

    lax.fori_loop(0, NCHF, body, 0)
    sl = lax.rem(NCHF - 1, 3)
    pltpu.make_async_copy(ob.at[sl],
                          out.at[pl.ds(ebase + (NCHF - 1) * CH, CH)],
                          sem_o).wait()

    @pl.when(wid < NEX)
    def _():
        st = EXTRA0 + wid * CH
        ld(0, st)
        ldw(0, st)
        gst(0)
        gw(0)
        compute(0, st)
        pltpu.async_copy(ob.at[0], out.at[pl.ds(st, CH)], sem_o)
        pltpu.make_async_copy(ob.at[0], out.at[pl.ds(st, CH)], sem_o).wait()


def _mmT(a, w):
    return lax.dot_general(a, w, (((1,), (1,)), ((), ())),
                           preferred_element_type=_f32)


def _leaky(v):
    return jnp.where(v >= 0, v, 0.01 * v)


def _tc2_body(accx2, sdis2, cnt2, x, c1, Wl1, bl1, Wr1, eW1a, eb1a, eW1b,
              eb1b, h1_o, dn1m_o):
    cnt = cnt2[0, :, 0:1] + cnt2[1, :, 0:1]
    accx = accx2[0] + accx2[1]
    invm = 1.0 / jnp.maximum(cnt, 1.0)
    inv = 1.0 / (cnt + 1.0)
    v = _mmT(accx * invm, Wl1[...]) + bl1[...][None, :] + _mmT(x[...], Wr1[...])
    h1_o[...] = _leaky(v)
    sdis = sdis2[0] + sdis2[1]
    t = _mmT(sdis, eW1a[...]) * inv + eb1a[...][None, :]
    u = _mmT(t, eW1b[...]) + eb1b[...][None, :]
    dn1m_o[...] = _leaky(u) - c1[...]


def _tc4_body(acch2, accd2, cnt2, h1, c1, Wl2, bl2, Wr2,
              eW2a, eb2a, eW2b, eb2b, Wfc, bfc,
              h2w_o, h2_o, ddn_o):
    cnt = cnt2[0, :, 0:1] + cnt2[1, :, 0:1]
    invm = 1.0 / jnp.maximum(cnt, 1.0)
    inv = 1.0 / (cnt + 1.0)
    acch = acch2[0] + acch2[1]
    v = _mmT(acch * invm, Wl2[...]) + bl2[...][None, :] + _mmT(h1[...], Wr2[...])
    h2 = _leaky(v)
    S = accd2[0] + accd2[1] + cnt * c1[...]
    t = _mmT(S, eW2a[...]) * inv + eb2a[...][None, :]
    u = _mmT(t, eW2b[...]) + eb2b[...][None, :]
    dn2 = _leaky(u)
    wA = Wfc[0, 0:64][None, :]
    wB = Wfc[0, 64:128][None, :]
    h2_o[...] = h2
    h2w_o[...] = h2 * wA
    ddn_o[...] = jnp.sum(dn2 * wB, axis=1, keepdims=True) + bfc[0]


def kernel(x, edge_index, dis, Wl1, bl1, Wr1, Wl2, bl2, Wr2,
           eW1a, eb1a, eW1b, eb1b, eW2a, eb2a, eW2b, eb2b, Wfc, bfc):
    ei = edge_index.astype(jnp.int32)
    z128 = jnp.zeros((N, DN), _f32)
    z16 = jnp.zeros((N, DE), _f32)

    c1 = _leaky(eb1a @ eW1b.T + eb1b)[None, :]
    c2 = _leaky(eb2a @ eW2b.T + eb2b)
    c2s = c2 @ Wfc[0, 64:128] + bfc[0]

    accx2, sdis2, cnt2 = _p1(ei, x, dis, z128, z16)

    NB = 2000
    grid = N // NB

    def _nblk(shape):
        nd = len(shape)
        blk = shape[:-2] + (NB, shape[-1])
        idx = lambda i: (0,) * (nd - 2) + (i, 0)
        return pl.BlockSpec(blk, idx)

    def _full(shape):
        return pl.BlockSpec(shape, lambda i: (0,) * len(shape))

    h1, dn1m = pl.pallas_call(
        _tc2_body,
        grid=(grid,),
        in_specs=[_nblk((NC, N, DN)), _nblk((NC, N, DE)), _nblk((NC, N, DE)),
                  _nblk((N, DN)), _full(c1.shape), _full(Wl1.shape),
                  _full(bl1.shape), _full(Wr1.shape), _full(eW1a.shape),
                  _full(eb1a.shape), _full(eW1b.shape), _full(eb1b.shape)],
        out_specs=[_nblk((N, HID)), _nblk((N, HID))],
        out_shape=[jax.ShapeDtypeStruct((N, HID), _f32),
                   jax.ShapeDtypeStruct((N, HID), _f32)],
    )(accx2, sdis2, cnt2, x, c1, Wl1, bl1, Wr1, eW1a, eb1a, eW1b, eb1b)

    acch2, accd2 = _p3(ei, h1, dn1m, z128)

    h2w, h2, ddn = pl.pallas_call(
        _tc4_body,
        grid=(grid,),
        in_specs=[_nblk((NC, N, HID)), _nblk((NC, N, HID)),
                  _nblk((NC, N, DE)), _nblk((N, HID)), _full(c1.shape),
                  _full(Wl2.shape), _full(bl2.shape), _full(Wr2.shape),
                  _full(eW2a.shape), _full(eb2a.shape), _full(eW2b.shape),
                  _full(eb2b.shape), _full(Wfc.shape), _full(bfc.shape)],
        out_specs=[_nblk((N, 64)), _nblk((N, 64)),
                   pl.BlockSpec((NB, 1), lambda i: (i, 0))],
        out_shape=[jax.ShapeDtypeStruct((N, 64), _f32),
                   jax.ShapeDtypeStruct((N, 64), _f32),
                   jax.ShapeDtypeStruct(((NLO + 1) * CH, 1), _f32)],
    )(acch2, accd2, cnt2, h1, c1, Wl2, bl2, Wr2,
      eW2a, eb2a, eW2b, eb2b, Wfc, bfc)

    c2v = jnp.full((16,), c2s, _f32)
    out = _p5(ei, h2w, h2, ddn[:, 0], c2v)
    return out[:, None]


# speedup vs baseline: 18.4050x; 1.0074x over previous
import functools

import jax
import jax.numpy as jnp
from jax import lax
from jax.experimental import pallas as pl
from jax.experimental.pallas import tpu as pltpu
from jax.experimental.pallas import tpu_sc as plsc

N = 10000
E = 320000
DN = 128
DE = 16
HID = 128
NC = 2
NS = 16
NW = NC * NS
NPT = 624

CH1 = 80
NCH1 = (E // NW) // CH1
CH = 128
NCHF = 78
SPT = NCHF * CH
EXTRA0 = NW * SPT
NEX = (E - EXTRA0) // CH
NLO = N // CH
LOTAIL = N - NLO * CH

_mesh = plsc.VectorSubcoreMesh(core_axis_name="c", subcore_axis_name="s")
_f32 = jnp.float32


def _fill_ones(ref, rows):
    def body(r, _):
        ref[r, pl.ds(0, 16)] = jnp.ones((16,), _f32)
        return 0
    lax.fori_loop(0, rows, body, 0)


def _tile_rows(s, fn):
    fn(s * NPT, NPT)

    @pl.when(s == NS - 1)
    def _():
        fn(NS * NPT, N - NS * NPT)


@functools.partial(
    pl.kernel, mesh=_mesh,
    compiler_params=pltpu.CompilerParams(use_tc_tiling_on_sc=False),
    out_type=[
        jax.ShapeDtypeStruct((NC, N, DN), _f32),
        jax.ShapeDtypeStruct((NC, N, 2 * DE), _f32),
    ],
    scratch_types=[
        pltpu.VMEM((3, CH1), jnp.int32),
        pltpu.VMEM((3, CH1), jnp.int32),
        pltpu.VMEM((2, CH1, DN), _f32),
        pltpu.VMEM((2, CH1, 2 * DE), _f32),
        pltpu.VMEM_SHARED((N, DN), _f32),
        pltpu.VMEM_SHARED((N, 2 * DE), _f32),
        pltpu.SemaphoreType.DMA,
        pltpu.SemaphoreType.DMA,
        pltpu.SemaphoreType.DMA,
    ],
)
def _p1(ei, x, dis, z128, z32,
        accx_o, sdc_o,
        rowb, colb, xrows, comb, accx_s, sdc_s,
        sem_l, sem_g, sem_s):
    c = lax.axis_index("c")
    s = lax.axis_index("s")
    wid = s * NC + c
    base = wid * NCH1 * CH1
    def fill(r, _):
        comb[0, r, pl.ds(DE, 16)] = jnp.ones((16,), _f32)
        comb[1, r, pl.ds(DE, 16)] = jnp.ones((16,), _f32)
        return 0
    lax.fori_loop(0, CH1, fill, 0)

    def init(o, n):
        pltpu.sync_copy(z128.at[pl.ds(o, n)], accx_s.at[pl.ds(o, n)])
        pltpu.sync_copy(z32.at[pl.ds(o, n)], sdc_s.at[pl.ds(o, n)])

    _tile_rows(s, init)
    plsc.subcore_barrier()

    def ld(slot, st):
        pltpu.async_copy(ei.at[0, pl.ds(st, CH1)], rowb.at[slot], sem_l)
        pltpu.async_copy(ei.at[1, pl.ds(st, CH1)], colb.at[slot], sem_l)

    def ldw(slot):
        pltpu.make_async_copy(ei.at[0, pl.ds(0, CH1)], rowb.at[slot],
                              sem_l).wait()
        pltpu.make_async_copy(ei.at[1, pl.ds(0, CH1)], colb.at[slot],
                              sem_l).wait()

    def sc_start(islot, dslot):
        pltpu.async_copy(xrows.at[dslot], accx_s.at[colb.at[islot]], sem_s,
                         add=True)
        pltpu.async_copy(comb.at[dslot], sdc_s.at[colb.at[islot]], sem_s,
                         add=True)

    def sc_wait(islot, dslot):
        pltpu.make_async_copy(xrows.at[dslot], accx_s.at[colb.at[islot]],
                              sem_s).wait()
        pltpu.make_async_copy(comb.at[dslot], sdc_s.at[colb.at[islot]],
                              sem_s).wait()

    def gst(islot, dslot, st):
        pltpu.async_copy(x.at[rowb.at[islot]], xrows.at[dslot], sem_g)
        pltpu.async_copy(dis.at[pl.ds(st, CH1)],
                         comb.at[dslot, pl.ds(0, CH1), pl.ds(0, DE)], sem_g)

    def gwt(islot, dslot):
        pltpu.make_async_copy(x.at[rowb.at[islot]], xrows.at[dslot],
                              sem_g).wait()
        pltpu.make_async_copy(dis.at[pl.ds(0, CH1)],
                              comb.at[dslot, pl.ds(0, CH1), pl.ds(0, DE)],
                              sem_g).wait()

    ld(0, base)
    ldw(0)
    gst(0, 0, base)
    ld(1, base + CH1)

    def body(j, _):
        sj = lax.rem(j, 3)
        sn = lax.rem(j + 1, 3)
        sp = lax.rem(j + 2, 3)
        dj = lax.rem(j, 2)
        dn = 1 - dj

        @pl.when(j > 0)
        def _():
            sc_wait(sp, dn)

        gwt(sj, dj)

        @pl.when(j + 1 < NCH1)
        def _():
            ldw(sn)
            gst(sn, dn, base + (j + 1) * CH1)

        @pl.when(j + 2 < NCH1)
        def _():
            ld(sp, base + (j + 2) * CH1)

        sc_start(sj, dj)
        return 0

    lax.fori_loop(0, NCH1, body, 0)
    sc_wait(lax.rem(NCH1 - 1, 3), lax.rem(NCH1 - 1, 2))
    plsc.subcore_barrier()

    def writeout(o, n):
        pltpu.sync_copy(accx_s.at[pl.ds(o, n)], accx_o.at[c, pl.ds(o, n)])
        pltpu.sync_copy(sdc_s.at[pl.ds(o, n)], sdc_o.at[c, pl.ds(o, n)])

    _tile_rows(s, writeout)


@functools.partial(
    pl.kernel, mesh=_mesh,
    compiler_params=pltpu.CompilerParams(use_tc_tiling_on_sc=False),
    out_type=[
        jax.ShapeDtypeStruct((NC, N, HID), _f32),
        jax.ShapeDtypeStruct((NC, N, HID), _f32),
    ],
    scratch_types=[
        pltpu.VMEM((3, CH), jnp.int32),
        pltpu.VMEM((3, CH), jnp.int32),
        pltpu.VMEM((2, CH, HID), _f32),
        pltpu.VMEM((LOTAIL,), jnp.int32),
        pltpu.VMEM((LOTAIL, HID), _f32),
        pltpu.VMEM_SHARED((N, HID), _f32),
        pltpu.SemaphoreType.DMA,
        pltpu.SemaphoreType.DMA,
        pltpu.SemaphoreType.DMA,
    ],
)
def _p3(ei, h1, dn1m, z128, acch_o, accd_o,
        rowb, colb, hrows, colb16, drows16, acc_s, sem_l, sem_g, sem_s):
    c = lax.axis_index("c")
    s = lax.axis_index("s")
    wid = s * NC + c
    ebase = wid * SPT

    def zero_table(o, n):
        pltpu.sync_copy(z128.at[pl.ds(o, n)], acc_s.at[pl.ds(o, n)])

    _tile_rows(s, zero_table)
    plsc.subcore_barrier()

    def ld(slot, st):
        pltpu.async_copy(ei.at[0, pl.ds(st, CH)], rowb.at[slot], sem_l)
        pltpu.async_copy(ei.at[1, pl.ds(st, CH)], colb.at[slot], sem_l)

    def ldw(slot):
        pltpu.make_async_copy(ei.at[0, pl.ds(0, CH)], rowb.at[slot],
                              sem_l).wait()
        pltpu.make_async_copy(ei.at[1, pl.ds(0, CH)], colb.at[slot],
                              sem_l).wait()

    ld(0, ebase)
    ldw(0)
    pltpu.async_copy(h1.at[rowb.at[0]], hrows.at[0], sem_g)
    ld(1, ebase + CH)

    def body(j, _):
        sj = lax.rem(j, 3)
        sn = lax.rem(j + 1, 3)
        sp = lax.rem(j + 2, 3)
        dj = lax.rem(j, 2)
        dn = 1 - dj

        @pl.when(j > 0)
        def _():
            pltpu.make_async_copy(hrows.at[dn], acc_s.at[colb.at[sp]],
                                  sem_s).wait()

        pltpu.make_async_copy(h1.at[rowb.at[sj]], hrows.at[dj], sem_g).wait()

        @pl.when(j + 1 < NCHF)
        def _():
            ldw(sn)
            pltpu.async_copy(h1.at[rowb.at[sn]], hrows.at[dn], sem_g)

        @pl.when(j + 2 < NCHF)
        def _():
            ld(sp, ebase + (j + 2) * CH)

        pltpu.async_copy(hrows.at[dj], acc_s.at[colb.at[sj]], sem_s,
                         add=True)
        return 0

    lax.fori_loop(0, NCHF, body, 0)
    pltpu.make_async_copy(hrows.at[lax.rem(NCHF - 1, 2)],
                          acc_s.at[colb.at[lax.rem(NCHF - 1, 3)]],
                          sem_s).wait()

    @pl.when(wid < NEX)
    def _():
        st = EXTRA0 + wid * CH
        ld(0, st)
        ldw(0)
        pltpu.async_copy(h1.at[rowb.at[0]], hrows.at[0], sem_g)
        pltpu.make_async_copy(h1.at[rowb.at[0]], hrows.at[0], sem_g).wait()
        pltpu.async_copy(hrows.at[0], acc_s.at[colb.at[0]], sem_s, add=True)
        pltpu.make_async_copy(hrows.at[0], acc_s.at[colb.at[0]],
                              sem_s).wait()

    plsc.subcore_barrier()

    def wr_acch(o, n):
        pltpu.sync_copy(acc_s.at[pl.ds(o, n)], acch_o.at[c, pl.ds(o, n)])
        pltpu.sync_copy(z128.at[pl.ds(o, n)], acc_s.at[pl.ds(o, n)])

    _tile_rows(s, wr_acch)
    plsc.subcore_barrier()

    for t in range((NLO + NW - 1) // NW):
        cid = wid + NW * t

        @pl.when(cid < NLO)
        def _():
            st = cid * CH
            pltpu.sync_copy(ei.at[1, pl.ds(st, CH)], colb.at[0])
            pltpu.sync_copy(dn1m.at[pl.ds(st, CH)], hrows.at[0])
            pltpu.async_copy(hrows.at[0], acc_s.at[colb.at[0]], sem_s,
                             add=True)
            pltpu.make_async_copy(hrows.at[0], acc_s.at[colb.at[0]],
                                  sem_s).wait()

    @pl.when(wid == NW - 1)
    def _():
        st = NLO * CH
        pltpu.sync_copy(ei.at[1, pl.ds(st, LOTAIL)], colb16)
        pltpu.sync_copy(dn1m.at[pl.ds(st, LOTAIL)], drows16)
        pltpu.async_copy(drows16, acc_s.at[colb16], sem_s, add=True)
        pltpu.make_async_copy(drows16, acc_s.at[colb16], sem_s).wait()

    plsc.subcore_barrier()
    _tile_rows(s, lambda o, n: pltpu.sync_copy(
        acc_s.at[pl.ds(o, n)], accd_o.at[c, pl.ds(o, n)]))


@functools.partial(
    pl.kernel, mesh=_mesh,
    out_type=jax.ShapeDtypeStruct((E,), _f32),
    compiler_params=pltpu.CompilerParams(use_tc_tiling_on_sc=False),
    scratch_types=[
        pltpu.VMEM((3, CH), jnp.int32),
        pltpu.VMEM((3, CH), jnp.int32),
        pltpu.VMEM((3, CH, 64), _f32),
        pltpu.VMEM((3, CH, 64), _f32),
        pltpu.VMEM((3, CH), _f32),
        pltpu.VMEM((3, CH), _f32),
        pltpu.VMEM((16,), _f32),
        pltpu.SemaphoreType.DMA,
        pltpu.SemaphoreType.DMA,
        pltpu.SemaphoreType.DMA,
    ],
)
def _p5(ei, h2w, h2, ddp, c2v, out,
        rowb, colb, rbuf, cbuf, ddb, ob, c2b, sem_l, sem_g, sem_o):
    c = lax.axis_index("c")
    s = lax.axis_index("s")
    wid = s * NC + c
    ebase = wid * SPT
    pltpu.sync_copy(c2v, c2b)
    vc2 = c2b[pl.ds(0, 16)]

    def ld(slot, st):
        pltpu.async_copy(ei.at[0, pl.ds(st, CH)], rowb.at[slot], sem_l)
        pltpu.async_copy(ei.at[1, pl.ds(st, CH)], colb.at[slot], sem_l)

        @pl.when(st < N)
        def _():
            pltpu.async_copy(ddp.at[pl.ds(st, CH)], ddb.at[slot], sem_l)

    def ldw(slot, st):
        pltpu.make_async_copy(ei.at[0, pl.ds(0, CH)], rowb.at[slot],
                              sem_l).wait()
        pltpu.make_async_copy(ei.at[1, pl.ds(0, CH)], colb.at[slot],
                              sem_l).wait()

        @pl.when(st < N)
        def _():
            pltpu.make_async_copy(ddp.at[pl.ds(0, CH)], ddb.at[slot],
                                  sem_l).wait()

    def gst(slot):
        pltpu.async_copy(h2w.at[rowb.at[slot]], rbuf.at[slot], sem_g)
        pltpu.async_copy(h2.at[colb.at[slot]], cbuf.at[slot], sem_g)

    def gw(slot):
        pltpu.make_async_copy(h2w.at[rowb.at[slot]], rbuf.at[slot],
                              sem_g).wait()
        pltpu.make_async_copy(h2.at[colb.at[slot]], cbuf.at[slot],
                              sem_g).wait()

    lane = lax.iota(jnp.int32, 16)
    p8 = lane ^ 8
    p4 = lane ^ 4
    p2 = lane ^ 2
    p1 = lane ^ 1

    def compute(sj, st):
        def group(g, _):
            def edot(e16, accv):
                e = g * 16 + e16
                v = rbuf[sj, e, pl.ds(0, 16)] * cbuf[sj, e, pl.ds(0, 16)]
                v = v + rbuf[sj, e, pl.ds(16, 16)] * cbuf[sj, e, pl.ds(16, 16)]
                v = v + rbuf[sj, e, pl.ds(32, 16)] * cbuf[sj, e, pl.ds(32, 16)]
                v = v + rbuf[sj, e, pl.ds(48, 16)] * cbuf[sj, e, pl.ds(48, 16)]
                v = v + v[p8]
                v = v + v[p4]
                v = v + v[p2]
                v = v + v[p1]
                return jnp.where(lane == e16, v, accv)

            accv = lax.fori_loop(0, 16, edot, jnp.zeros((16,), _f32))
            eid = st + g * 16 + lane
            ddv = jnp.where(eid < N, ddb[sj, pl.ds(g * 16, 16)], vc2)
            ob[sj, pl.ds(g * 16, 16)] = accv + ddv
            return 0

        lax.fori_loop(0, CH // 16, group, 0)

    ld(0, ebase)
    ldw(0, ebase)
    gst(0)
    ld(1, ebase + CH)

    def body(j, _):
        sj = lax.rem(j, 3)
        sn = lax.rem(j + 1, 3)
        sp = lax.rem(j + 2, 3)

        @pl.when(j > 0)
        def _():
            pltpu.make_async_copy(
                ob.at[sp], out.at[pl.ds(ebase + (j - 1) * CH, CH)],
                sem_o).wait()

        gw(sj)

        @pl.when(j + 1 < NCHF)
        def _():
            ldw(sn, ebase + (j + 1) * CH)
            gst(sn)

        @pl.when(j + 2 < NCHF)
        def _():
            ld(sp, ebase + (j + 2) * CH)

        compute(sj, ebase + j * CH)
        pltpu.async_copy(ob.at[sj], out.at[pl.ds(ebase + j * CH, CH)], sem_o)
        return 0

    lax.fori_loop(0, NCHF, body, 0)
    sl = lax.rem(NCHF - 1, 3)
    pltpu.make_async_copy(ob.at[sl],
                          out.at[pl.ds(ebase + (NCHF - 1) * CH, CH)],
                          sem_o).wait()

    @pl.when(wid < NEX)
    def _():
        st = EXTRA0 + wid * CH
        ld(0, st)
        ldw(0, st)
        gst(0)
        gw(0)
        compute(0, st)
        pltpu.async_copy(ob.at[0], out.at[pl.ds(st, CH)], sem_o)
        pltpu.make_async_copy(ob.at[0], out.at[pl.ds(st, CH)], sem_o).wait()


def _mmT(a, w):
    return lax.dot_general(a, w, (((1,), (1,)), ((), ())),
                           preferred_element_type=_f32)


def _leaky(v):
    return jnp.where(v >= 0, v, 0.01 * v)


def _tc2_body(accx2, sdc2, x, c1, Wl1, bl1, Wr1, eW1a, eb1a, eW1b,
              eb1b, h1_o, dn1m_o):
    cnt = sdc2[0, :, DE:DE + 1] + sdc2[1, :, DE:DE + 1]
    accx = accx2[0] + accx2[1]
    invm = 1.0 / jnp.maximum(cnt, 1.0)
    inv = 1.0 / (cnt + 1.0)
    v = _mmT(accx * invm, Wl1[...]) + bl1[...][None, :] + _mmT(x[...], Wr1[...])
    h1_o[...] = _leaky(v)
    sdis = sdc2[0, :, 0:DE] + sdc2[1, :, 0:DE]
    t = _mmT(sdis, eW1a[...]) * inv + eb1a[...][None, :]
    u = _mmT(t, eW1b[...]) + eb1b[...][None, :]
    dn1m_o[...] = _leaky(u) - c1[...]


def _tc4_body(acch2, accd2, sdc2, h1, c1, Wl2, bl2, Wr2,
              eW2a, eb2a, eW2b, eb2b, Wfc, bfc,
              h2w_o, h2_o, ddn_o):
    cnt = sdc2[0, :, DE:DE + 1] + sdc2[1, :, DE:DE + 1]
    invm = 1.0 / jnp.maximum(cnt, 1.0)
    inv = 1.0 / (cnt + 1.0)
    acch = acch2[0] + acch2[1]
    v = _mmT(acch * invm, Wl2[...]) + bl2[...][None, :] + _mmT(h1[...], Wr2[...])
    h2 = _leaky(v)
    S = accd2[0] + accd2[1] + cnt * c1[...]
    t = _mmT(S, eW2a[...]) * inv + eb2a[...][None, :]
    u = _mmT(t, eW2b[...]) + eb2b[...][None, :]
    dn2 = _leaky(u)
    wA = Wfc[0, 0:64][None, :]
    wB = Wfc[0, 64:128][None, :]
    h2_o[...] = h2
    h2w_o[...] = h2 * wA
    ddn_o[...] = jnp.sum(dn2 * wB, axis=1, keepdims=True) + bfc[0]


def kernel(x, edge_index, dis, Wl1, bl1, Wr1, Wl2, bl2, Wr2,
           eW1a, eb1a, eW1b, eb1b, eW2a, eb2a, eW2b, eb2b, Wfc, bfc):
    ei = edge_index.astype(jnp.int32)
    z128 = jnp.zeros((N, DN), _f32)
    z16 = jnp.zeros((N, DE), _f32)
    z32 = jnp.zeros((N, 2 * DE), _f32)

    c1 = _leaky(eb1a @ eW1b.T + eb1b)[None, :]
    c2 = _leaky(eb2a @ eW2b.T + eb2b)
    c2s = c2 @ Wfc[0, 64:128] + bfc[0]

    accx2, sdc2 = _p1(ei, x, dis, z128, z32)

    NB = 2000
    grid = N // NB

    def _nblk(shape):
        nd = len(shape)
        blk = shape[:-2] + (NB, shape[-1])
        idx = lambda i: (0,) * (nd - 2) + (i, 0)
        return pl.BlockSpec(blk, idx)

    def _full(shape):
        return pl.BlockSpec(shape, lambda i: (0,) * len(shape))

    h1, dn1m = pl.pallas_call(
        _tc2_body,
        grid=(grid,),
        in_specs=[_nblk((NC, N, DN)), _nblk((NC, N, 2 * DE)),
                  _nblk((N, DN)), _full(c1.shape), _full(Wl1.shape),
                  _full(bl1.shape), _full(Wr1.shape), _full(eW1a.shape),
                  _full(eb1a.shape), _full(eW1b.shape), _full(eb1b.shape)],
        out_specs=[_nblk((N, HID)), _nblk((N, HID))],
        out_shape=[jax.ShapeDtypeStruct((N, HID), _f32),
                   jax.ShapeDtypeStruct((N, HID), _f32)],
    )(accx2, sdc2, x, c1, Wl1, bl1, Wr1, eW1a, eb1a, eW1b, eb1b)

    acch2, accd2 = _p3(ei, h1, dn1m, z128)

    h2w, h2, ddn = pl.pallas_call(
        _tc4_body,
        grid=(grid,),
        in_specs=[_nblk((NC, N, HID)), _nblk((NC, N, HID)),
                  _nblk((NC, N, 2 * DE)), _nblk((N, HID)), _full(c1.shape),
                  _full(Wl2.shape), _full(bl2.shape), _full(Wr2.shape),
                  _full(eW2a.shape), _full(eb2a.shape), _full(eW2b.shape),
                  _full(eb2b.shape), _full(Wfc.shape), _full(bfc.shape)],
        out_specs=[_nblk((N, 64)), _nblk((N, 64)),
                   pl.BlockSpec((NB, 1), lambda i: (i, 0))],
        out_shape=[jax.ShapeDtypeStruct((N, 64), _f32),
                   jax.ShapeDtypeStruct((N, 64), _f32),
                   jax.ShapeDtypeStruct(((NLO + 1) * CH, 1), _f32)],
    )(acch2, accd2, sdc2, h1, c1, Wl2, bl2, Wr2,
      eW2a, eb2a, eW2b, eb2b, Wfc, bfc)

    c2v = jnp.full((16,), c2s, _f32)
    out = _p5(ei, h2w, h2, ddn[:, 0], c2v)
    return out[:, None]
